# Initial kernel scaffold; baseline (speedup 1.0000x reference)
import jax
import jax.numpy as jnp
from jax.experimental import pallas as pl


def kernel(agent_states, edge_index, params):
    raise NotImplementedError("write your pallas kernel here")


# speedup vs baseline: 13.7554x; 13.7554x over previous
import functools

import jax
import jax.numpy as jnp
from jax import lax
from jax.experimental import pallas as pl
from jax.experimental.pallas import tpu as pltpu
from jax.experimental.pallas import tpu_sc as plsc

N = 4096
E = 262144
D = 128
H = 4
C = D // H


_PASS_ROWS = 128
_PASSES_PER_CORE = (N // _PASS_ROWS) // 2
_BLK = _PASS_ROWS * N
_DUMP = _BLK
_TILE_SLICE = _BLK // 16
_TILE_EDGES = E // 16
_CHUNK = 4096


def _cnt_body(src_hbm, dst_hbm, ones_hbm, zeros_hbm, out_hbm,
              src_v, dst_v, idx_v, ones_v, zeros_v, acc):
    cid = lax.axis_index("c")
    sid = lax.axis_index("s")
    pltpu.sync_copy(ones_hbm, ones_v)
    pltpu.sync_copy(zeros_hbm, zeros_v)
    for p in range(_PASSES_PER_CORE):
        base_row = (cid * _PASSES_PER_CORE + p) * _PASS_ROWS
        pltpu.sync_copy(zeros_v, acc.at[pl.ds(sid * _TILE_SLICE, _TILE_SLICE)])

        @pl.when(sid == 0)
        def _():
            pltpu.sync_copy(zeros_v.at[pl.ds(0, 16)], acc.at[pl.ds(_BLK, 16)])

        plsc.subcore_barrier()
        for ch in range(_TILE_EDGES // _CHUNK):
            off = sid * _TILE_EDGES + ch * _CHUNK
            pltpu.sync_copy(dst_hbm.at[pl.ds(off, _CHUNK)], dst_v)
            pltpu.sync_copy(src_hbm.at[pl.ds(off, _CHUNK)], src_v)

            def body(i, carry):
                d = dst_v[pl.ds(i * 16, 16)]
                s = src_v[pl.ds(i * 16, 16)]
                rel = d - base_row
                m = (rel >= 0) & (rel < _PASS_ROWS)
                idx_v[pl.ds(i * 16, 16)] = jnp.where(m, rel * N + s, _DUMP)
                return carry

            lax.fori_loop(0, _CHUNK // 16, body, 0)
            pltpu.sync_copy(ones_v, acc.at[idx_v], add=True)
        plsc.subcore_barrier()
        out_base = base_row * N + sid * _TILE_SLICE
        pltpu.sync_copy(acc.at[pl.ds(sid * _TILE_SLICE, _TILE_SLICE)],
                        out_hbm.at[pl.ds(out_base, _TILE_SLICE)])
        plsc.subcore_barrier()


def _build_cnt(edge_index):
    src = edge_index[0]
    dst = edge_index[1]
    ones = jnp.ones((_CHUNK,), jnp.float32)
    zeros = jnp.zeros((_TILE_SLICE,), jnp.float32)
    mesh = plsc.VectorSubcoreMesh(core_axis_name="c", subcore_axis_name="s")
    k = pl.kernel(
        _cnt_body,
        out_type=jax.ShapeDtypeStruct((N * N,), jnp.float32),
        mesh=mesh,
        scratch_types=[
            pltpu.VMEM((_CHUNK,), jnp.int32),
            pltpu.VMEM((_CHUNK,), jnp.int32),
            pltpu.VMEM((_CHUNK,), jnp.int32),
            pltpu.VMEM((_CHUNK,), jnp.float32),
            pltpu.VMEM((_TILE_SLICE,), jnp.float32),
            pltpu.VMEM_SHARED((_BLK + 16,), jnp.float32),
        ],
    )
    return k(src, dst, ones, zeros).reshape(N, N)


_R_PRE = 512
_R_GAT = 256
_R_MHA = 256


def _dot(a, b):
    return jnp.dot(a, b, preferred_element_type=jnp.float32)


def _pre_body(n_add, *refs):
    i = pl.program_id(0)
    x_refs = refs[:n_add]
    w_ref, acat_ref = refs[n_add], refs[n_add + 1]
    if n_add > 1:
        xp_ref, a_ref, at_ref, m_ref, x_ref = refs[n_add + 2:]
    else:
        xp_ref, a_ref, at_ref, m_ref = refs[n_add + 2:]
    x = x_refs[0][...]
    for r in x_refs[1:]:
        x = x + r[...]
    if n_add > 1:
        x_ref[...] = x
    xp = _dot(x, w_ref[...])
    a = _dot(xp, acat_ref[...])
    xp_ref[...] = xp
    a_ref[...] = a
    at_ref[...] = a.T
    blk_max = jnp.max(a, axis=0, keepdims=True)

    @pl.when(i == 0)
    def _():
        m_ref[...] = blk_max

    @pl.when(i != 0)
    def _():
        m_ref[...] = jnp.maximum(m_ref[...], blk_max)


def _gat_pre(xs, w, a_cat):
    n_add = len(xs)
    grid = (N // _R_PRE,)
    row_spec = pl.BlockSpec((_R_PRE, D), lambda i: (i, 0))
    out_specs = [
        pl.BlockSpec((_R_PRE, D), lambda i: (i, 0)),
        pl.BlockSpec((_R_PRE, 8), lambda i: (i, 0)),
        pl.BlockSpec((8, _R_PRE), lambda i: (0, i)),
        pl.BlockSpec((1, 8), lambda i: (0, 0)),
    ]
    out_shape = [
        jax.ShapeDtypeStruct((N, D), jnp.float32),
        jax.ShapeDtypeStruct((N, 8), jnp.float32),
        jax.ShapeDtypeStruct((8, N), jnp.float32),
        jax.ShapeDtypeStruct((1, 8), jnp.float32),
    ]
    if n_add > 1:
        out_specs.append(pl.BlockSpec((_R_PRE, D), lambda i: (i, 0)))
        out_shape.append(jax.ShapeDtypeStruct((N, D), jnp.float32))
    out = pl.pallas_call(
        functools.partial(_pre_body, n_add),
        grid=grid,
        in_specs=[row_spec] * n_add + [
            pl.BlockSpec((D, D), lambda i: (0, 0)),
            pl.BlockSpec((D, 8), lambda i: (0, 0)),
        ],
        out_specs=out_specs,
        out_shape=out_shape,
    )(*xs, w, a_cat)
    return out


def _lrelu(t):
    return jnp.where(t > 0, t, 0.2 * t)


def _gat_body(cnt_ref, xp_ref, a_ref, at_ref, m_ref, res_ref,
              bias_ref, g_ref, b_ref, out_ref):
    i = pl.program_id(0)
    ad = a_ref[...][:, 4:8]
    m_row = m_ref[...][:, 0:4]
    ast = at_ref[...][0:4, :]
    m_col = jnp.max(ast, axis=1, keepdims=True)
    shift = _lrelu(m_row + ad)
    ed_a = jnp.exp(ad + m_row - shift)
    ed_b = jnp.exp(0.2 * (ad + m_row) - shift)
    es_a = jnp.exp(ast - m_col)
    es_b = jnp.exp(0.2 * (ast - m_col))

    rows = lax.broadcasted_iota(jnp.int32, (_R_GAT, N), 0) + i * _R_GAT
    cols = lax.broadcasted_iota(jnp.int32, (_R_GAT, N), 1)
    cnt = cnt_ref[...] + jnp.where(rows == cols, 1.0, 0.0)

    outs = []
    for h in range(H):
        z = ast[h:h + 1, :] + ad[:, h:h + 1]
        w = cnt * jnp.where(
            z > 0,
            ed_a[:, h:h + 1] * es_a[h:h + 1, :],
            ed_b[:, h:h + 1] * es_b[h:h + 1, :],
        )
        den = jnp.sum(w, axis=1, keepdims=True)
        num = _dot(w, xp_ref[...][:, h * C:(h + 1) * C])
        outs.append(num / (den + 1e-16))
    out = jnp.concatenate(outs, axis=1) + bias_ref[...]
    mu = jnp.mean(out, axis=1, keepdims=True)
    var = jnp.mean((out - mu) ** 2, axis=1, keepdims=True)
    out = (out - mu) * lax.rsqrt(var + 1e-5) * g_ref[...] + b_ref[...]
    out_ref[...] = out + res_ref[...]


def _gat_dense(cnt, xp, a_rows, a_t, m, resid, bias, ln_g, ln_b):
    grid = (N // _R_GAT,)
    return pl.pallas_call(
        _gat_body,
        grid=grid,
        in_specs=[
            pl.BlockSpec((_R_GAT, N), lambda i: (i, 0)),
            pl.BlockSpec((N, D), lambda i: (0, 0)),
            pl.BlockSpec((_R_GAT, 8), lambda i: (i, 0)),
            pl.BlockSpec((8, N), lambda i: (0, 0)),
            pl.BlockSpec((1, 8), lambda i: (0, 0)),
            pl.BlockSpec((_R_GAT, D), lambda i: (i, 0)),
            pl.BlockSpec((1, D), lambda i: (0, 0)),
            pl.BlockSpec((1, D), lambda i: (0, 0)),
            pl.BlockSpec((1, D), lambda i: (0, 0)),
        ],
        out_specs=pl.BlockSpec((_R_GAT, D), lambda i: (i, 0)),
        out_shape=jax.ShapeDtypeStruct((N, D), jnp.float32),
    )(cnt, xp, a_rows, a_t, m, resid, bias, ln_g, ln_b)


def _mlp_body(x_ref, ew1, eb1, ew2, eb2, dw1, db1, dw2, db2, qw, qb, qkv_ref):
    x = x_ref[...]
    h1 = jnp.maximum(_dot(x, ew1[...]) + eb1[...], 0.0)
    msg = _dot(h1, ew2[...]) + eb2[...]
    d1 = jnp.maximum(_dot(msg, dw1[...]) + db1[...], 0.0)
    dec = _dot(d1, dw2[...]) + db2[...]
    qkv_ref[...] = _dot(dec, qw[...]) + qb[...]


def _mlp(x, p):
    grid = (N // _R_PRE,)
    full = lambda a: pl.BlockSpec(a.shape, lambda i: (0,) * a.ndim)
    args = [p['enc_W1'], p['enc_b1'].reshape(1, -1), p['enc_W2'],
            p['enc_b2'].reshape(1, -1), p['dec_W1'], p['dec_b1'].reshape(1, -1),
            p['dec_W2'], p['dec_b2'].reshape(1, -1), p['mha_in_W'],
            p['mha_in_b'].reshape(1, -1)]
    return pl.pallas_call(
        _mlp_body,
        grid=grid,
        in_specs=[pl.BlockSpec((_R_PRE, D), lambda i: (i, 0))] +
                 [full(a) for a in args],
        out_specs=pl.BlockSpec((_R_PRE, 3 * D), lambda i: (i, 0)),
        out_shape=jax.ShapeDtypeStruct((N, 3 * D), jnp.float32),
    )(x, *args)


def _mha_body(qt_ref, kv_ref, o_ref):
    outs = []
    for h in range(H):
        q = qt_ref[...][:, h * C:(h + 1) * C]
        k = kv_ref[...][:, D + h * C:D + (h + 1) * C]
        v = kv_ref[...][:, 2 * D + h * C:2 * D + (h + 1) * C]
        scores = lax.dot_general(q, k, (((1,), (1,)), ((), ())),
                                 preferred_element_type=jnp.float32)
        scores = scores * (1.0 / jnp.sqrt(float(C)))
        mx = jnp.max(scores, axis=1, keepdims=True)
        p = jnp.exp(scores - mx)
        s = jnp.sum(p, axis=1, keepdims=True)
        outs.append(_dot(p / s, v))
    o_ref[...] = jnp.concatenate(outs, axis=1)


def _mha(qkv):
    grid = (N // _R_MHA,)
    return pl.pallas_call(
        _mha_body,
        grid=grid,
        in_specs=[
            pl.BlockSpec((_R_MHA, 3 * D), lambda i: (i, 0)),
            pl.BlockSpec((N, 3 * D), lambda i: (0, 0)),
        ],
        out_specs=pl.BlockSpec((_R_MHA, D), lambda i: (i, 0)),
        out_shape=jax.ShapeDtypeStruct((N, D), jnp.float32),
    )(qkv, qkv)


def _final_body(o_ref, st_ref, ow, ob, gw_s, gw_a, gb1, gw2, gb2, pw, pb,
                out_ref):
    st = st_ref[...]
    agg = _dot(o_ref[...], ow[...]) + ob[...]
    g1 = jnp.maximum(_dot(st, gw_s[...]) + _dot(agg, gw_a[...]) + gb1[...], 0.0)
    logit = jnp.sum(g1 * gw2[...], axis=1, keepdims=True) + gb2[...]
    strength = 1.0 / (1.0 + jnp.exp(-logit))
    out_ref[...] = _dot(agg * strength, pw[...]) + pb[...] + st


def _final(o, states, p):
    grid = (N // _R_PRE,)
    full = lambda a: pl.BlockSpec(a.shape, lambda i: (0,) * a.ndim)
    args = [p['mha_out_W'], p['mha_out_b'].reshape(1, -1),
            p['gate_W1'][:D], p['gate_W1'][D:], p['gate_b1'].reshape(1, -1),
            p['gate_W2'].reshape(1, -1), p['gate_b2'].reshape(1, 1),
            p['proj_W'], p['proj_b'].reshape(1, -1)]
    return pl.pallas_call(
        _final_body,
        grid=grid,
        in_specs=[pl.BlockSpec((_R_PRE, D), lambda i: (i, 0)),
                  pl.BlockSpec((_R_PRE, D), lambda i: (i, 0))] +
                 [full(a) for a in args],
        out_specs=pl.BlockSpec((_R_PRE, D), lambda i: (i, 0)),
        out_shape=jax.ShapeDtypeStruct((N, D), jnp.float32),
    )(o, states, *args)


def _att_mat(gp):
    head_of_col = jnp.arange(D)[:, None] // C == jnp.arange(H)[None, :]
    a_src = jnp.where(head_of_col, gp['att_src'].reshape(-1)[:, None], 0.0)
    a_dst = jnp.where(head_of_col, gp['att_dst'].reshape(-1)[:, None], 0.0)
    return jnp.concatenate([a_src, a_dst], axis=1).astype(jnp.float32)


def _tc_pipeline(agent_states, params, cnt):
    p = params
    role_full = jnp.tile(p['role_emb'], (1, 4))
    x = None
    for l in range(2):
        gp = p['gat'][l]
        if l == 0:
            xs = [agent_states, p['agent_emb'], role_full]
            xp, a_rows, a_t, m, x0 = _gat_pre(xs, gp['W'], _att_mat(gp))
            resid = x0
        else:
            xp, a_rows, a_t, m = _gat_pre([x], gp['W'], _att_mat(gp))
            resid = x
        x = _gat_dense(cnt, xp, a_rows, a_t, m, resid,
                       gp['bias'].reshape(1, -1), p['ln_g'][l].reshape(1, -1),
                       p['ln_b'][l].reshape(1, -1))
    qkv = _mlp(x, p)
    o = _mha(qkv)
    return _final(o, agent_states, p)


def kernel(agent_states, edge_index, params):
    cnt = _build_cnt(edge_index)
    return _tc_pipeline(agent_states, params, cnt)


# speedup vs baseline: 25.9400x; 1.8858x over previous
import functools

import jax
import jax.numpy as jnp
from jax import lax
from jax.experimental import pallas as pl
from jax.experimental.pallas import tpu as pltpu
from jax.experimental.pallas import tpu_sc as plsc

N = 4096
E = 262144
D = 128
H = 4
C = D // H


_ROWS = 16
_ROUNDS = N // (32 * _ROWS)
_ACC = _ROWS * N
_CHUNK = 16384
_UNROLL = 8


def _cnt_body(src_hbm, dst_hbm, zeros_hbm, out_hbm, e_v, e2_v, acc_v, pk_sh):
    cid = lax.axis_index("c")
    sid = lax.axis_index("s")
    wid = sid * 2 + cid
    base_e = sid * (E // 16)
    pltpu.sync_copy(dst_hbm.at[pl.ds(base_e, E // 16)], e_v)
    pltpu.sync_copy(src_hbm.at[pl.ds(base_e, E // 16)], e2_v)

    def pack_body(i, carry):
        d = e_v[pl.ds(i * 16, 16)]
        s = e2_v[pl.ds(i * 16, 16)]
        e_v[pl.ds(i * 16, 16)] = d * N + s
        return carry

    lax.fori_loop(0, (E // 16) // 16, pack_body, 0)
    pltpu.sync_copy(e_v, pk_sh.at[pl.ds(base_e, E // 16)])
    plsc.subcore_barrier()
    def round_body(r, carry):
        row0 = r * (32 * _ROWS) + wid * _ROWS
        base_flat = row0 * N
        pltpu.sync_copy(zeros_hbm, acc_v)

        def chunk_body(ch, c2):
            pltpu.sync_copy(pk_sh.at[pl.ds(ch * _CHUNK, _CHUNK)], e_v)

            def body(i, c3):
                for u in range(_UNROLL):
                    off = (i * _UNROLL + u) * 16
                    t = e_v[pl.ds(off, 16)] - base_flat
                    m = (t >= 0) & (t < _ACC)
                    cnts, last = plsc.scan_count(t, m)
                    plsc.addupdate_scatter(acc_v, [t],
                                           cnts.astype(jnp.float32), mask=last)
                return c3

            lax.fori_loop(0, _CHUNK // 16 // _UNROLL, body, 0)
            return c2

        lax.fori_loop(0, E // _CHUNK, chunk_body, 0)
        pltpu.sync_copy(acc_v, out_hbm.at[pl.ds(base_flat, _ACC)])
        return carry

    lax.fori_loop(0, _ROUNDS, round_body, 0)


def _build_cnt(edge_index):
    src = edge_index[0]
    dst = edge_index[1]
    zeros = jnp.zeros((_ACC,), jnp.float32)
    mesh = plsc.VectorSubcoreMesh(core_axis_name="c", subcore_axis_name="s")
    k = pl.kernel(
        _cnt_body,
        out_type=jax.ShapeDtypeStruct((N * N,), jnp.float32),
        mesh=mesh,
        compiler_params=pltpu.CompilerParams(needs_layout_passes=False),
        scratch_types=[
            pltpu.VMEM((_CHUNK,), jnp.int32),
            pltpu.VMEM((_CHUNK,), jnp.int32),
            pltpu.VMEM((_ACC,), jnp.float32),
            pltpu.VMEM_SHARED((E,), jnp.int32),
        ],
    )
    return k(src, dst, zeros).reshape(N, N)


_R_PRE = 512
_R_GAT = 256
_R_MHA = 256


def _dot(a, b):
    return jnp.dot(a, b, preferred_element_type=jnp.float32)


def _pre_body(n_add, *refs):
    i = pl.program_id(0)
    x_refs = refs[:n_add]
    w_ref, acat_ref = refs[n_add], refs[n_add + 1]
    if n_add > 1:
        xp_ref, a_ref, at_ref, m_ref, x_ref = refs[n_add + 2:]
    else:
        xp_ref, a_ref, at_ref, m_ref = refs[n_add + 2:]
    x = x_refs[0][...]
    for r in x_refs[1:]:
        x = x + r[...]
    if n_add > 1:
        x_ref[...] = x
    xp = _dot(x, w_ref[...])
    a = _dot(xp, acat_ref[...])
    xp_ref[...] = xp
    a_ref[...] = a
    at_ref[...] = a.T
    blk_max = jnp.max(a, axis=0, keepdims=True)

    @pl.when(i == 0)
    def _():
        m_ref[...] = blk_max

    @pl.when(i != 0)
    def _():
        m_ref[...] = jnp.maximum(m_ref[...], blk_max)


def _gat_pre(xs, w, a_cat):
    n_add = len(xs)
    grid = (N // _R_PRE,)
    row_spec = pl.BlockSpec((_R_PRE, D), lambda i: (i, 0))
    out_specs = [
        pl.BlockSpec((_R_PRE, D), lambda i: (i, 0)),
        pl.BlockSpec((_R_PRE, 8), lambda i: (i, 0)),
        pl.BlockSpec((8, _R_PRE), lambda i: (0, i)),
        pl.BlockSpec((1, 8), lambda i: (0, 0)),
    ]
    out_shape = [
        jax.ShapeDtypeStruct((N, D), jnp.float32),
        jax.ShapeDtypeStruct((N, 8), jnp.float32),
        jax.ShapeDtypeStruct((8, N), jnp.float32),
        jax.ShapeDtypeStruct((1, 8), jnp.float32),
    ]
    if n_add > 1:
        out_specs.append(pl.BlockSpec((_R_PRE, D), lambda i: (i, 0)))
        out_shape.append(jax.ShapeDtypeStruct((N, D), jnp.float32))
    out = pl.pallas_call(
        functools.partial(_pre_body, n_add),
        grid=grid,
        in_specs=[row_spec] * n_add + [
            pl.BlockSpec((D, D), lambda i: (0, 0)),
            pl.BlockSpec((D, 8), lambda i: (0, 0)),
        ],
        out_specs=out_specs,
        out_shape=out_shape,
    )(*xs, w, a_cat)
    return out


def _lrelu(t):
    return jnp.where(t > 0, t, 0.2 * t)


def _gat_body(cnt_ref, xp_ref, a_ref, at_ref, m_ref, res_ref,
              bias_ref, g_ref, b_ref, out_ref):
    i = pl.program_id(0)
    ad = a_ref[...][:, 4:8]
    m_row = m_ref[...][:, 0:4]
    ast = at_ref[...][0:4, :]
    m_col = jnp.max(ast, axis=1, keepdims=True)
    shift = _lrelu(m_row + ad)
    ed_a = jnp.exp(ad + m_row - shift)
    ed_b = jnp.exp(0.2 * (ad + m_row) - shift)
    es_a = jnp.exp(ast - m_col)
    es_b = jnp.exp(0.2 * (ast - m_col))

    rows = lax.broadcasted_iota(jnp.int32, (_R_GAT, N), 0) + i * _R_GAT
    cols = lax.broadcasted_iota(jnp.int32, (_R_GAT, N), 1)
    cnt = cnt_ref[...] + jnp.where(rows == cols, 1.0, 0.0)

    outs = []
    for h in range(H):
        z = ast[h:h + 1, :] + ad[:, h:h + 1]
        w = cnt * jnp.where(
            z > 0,
            ed_a[:, h:h + 1] * es_a[h:h + 1, :],
            ed_b[:, h:h + 1] * es_b[h:h + 1, :],
        )
        den = jnp.sum(w, axis=1, keepdims=True)
        num = _dot(w, xp_ref[...][:, h * C:(h + 1) * C])
        outs.append(num / (den + 1e-16))
    out = jnp.concatenate(outs, axis=1) + bias_ref[...]
    mu = jnp.mean(out, axis=1, keepdims=True)
    var = jnp.mean((out - mu) ** 2, axis=1, keepdims=True)
    out = (out - mu) * lax.rsqrt(var + 1e-5) * g_ref[...] + b_ref[...]
    out_ref[...] = out + res_ref[...]


def _gat_dense(cnt, xp, a_rows, a_t, m, resid, bias, ln_g, ln_b):
    grid = (N // _R_GAT,)
    return pl.pallas_call(
        _gat_body,
        grid=grid,
        in_specs=[
            pl.BlockSpec((_R_GAT, N), lambda i: (i, 0)),
            pl.BlockSpec((N, D), lambda i: (0, 0)),
            pl.BlockSpec((_R_GAT, 8), lambda i: (i, 0)),
            pl.BlockSpec((8, N), lambda i: (0, 0)),
            pl.BlockSpec((1, 8), lambda i: (0, 0)),
            pl.BlockSpec((_R_GAT, D), lambda i: (i, 0)),
            pl.BlockSpec((1, D), lambda i: (0, 0)),
            pl.BlockSpec((1, D), lambda i: (0, 0)),
            pl.BlockSpec((1, D), lambda i: (0, 0)),
        ],
        out_specs=pl.BlockSpec((_R_GAT, D), lambda i: (i, 0)),
        out_shape=jax.ShapeDtypeStruct((N, D), jnp.float32),
    )(cnt, xp, a_rows, a_t, m, resid, bias, ln_g, ln_b)


def _mlp_body(x_ref, ew1, eb1, ew2, eb2, dw1, db1, dw2, db2, qw, qb, qkv_ref):
    x = x_ref[...]
    h1 = jnp.maximum(_dot(x, ew1[...]) + eb1[...], 0.0)
    msg = _dot(h1, ew2[...]) + eb2[...]
    d1 = jnp.maximum(_dot(msg, dw1[...]) + db1[...], 0.0)
    dec = _dot(d1, dw2[...]) + db2[...]
    qkv_ref[...] = _dot(dec, qw[...]) + qb[...]


def _mlp(x, p):
    grid = (N // _R_PRE,)
    full = lambda a: pl.BlockSpec(a.shape, lambda i: (0,) * a.ndim)
    args = [p['enc_W1'], p['enc_b1'].reshape(1, -1), p['enc_W2'],
            p['enc_b2'].reshape(1, -1), p['dec_W1'], p['dec_b1'].reshape(1, -1),
            p['dec_W2'], p['dec_b2'].reshape(1, -1), p['mha_in_W'],
            p['mha_in_b'].reshape(1, -1)]
    return pl.pallas_call(
        _mlp_body,
        grid=grid,
        in_specs=[pl.BlockSpec((_R_PRE, D), lambda i: (i, 0))] +
                 [full(a) for a in args],
        out_specs=pl.BlockSpec((_R_PRE, 3 * D), lambda i: (i, 0)),
        out_shape=jax.ShapeDtypeStruct((N, 3 * D), jnp.float32),
    )(x, *args)


def _mha_body(qt_ref, kv_ref, o_ref):
    outs = []
    for h in range(H):
        q = qt_ref[...][:, h * C:(h + 1) * C]
        k = kv_ref[...][:, D + h * C:D + (h + 1) * C]
        v = kv_ref[...][:, 2 * D + h * C:2 * D + (h + 1) * C]
        scores = lax.dot_general(q, k, (((1,), (1,)), ((), ())),
                                 preferred_element_type=jnp.float32)
        scores = scores * (1.0 / jnp.sqrt(float(C)))
        mx = jnp.max(scores, axis=1, keepdims=True)
        p = jnp.exp(scores - mx)
        s = jnp.sum(p, axis=1, keepdims=True)
        outs.append(_dot(p / s, v))
    o_ref[...] = jnp.concatenate(outs, axis=1)


def _mha(qkv):
    grid = (N // _R_MHA,)
    return pl.pallas_call(
        _mha_body,
        grid=grid,
        in_specs=[
            pl.BlockSpec((_R_MHA, 3 * D), lambda i: (i, 0)),
            pl.BlockSpec((N, 3 * D), lambda i: (0, 0)),
        ],
        out_specs=pl.BlockSpec((_R_MHA, D), lambda i: (i, 0)),
        out_shape=jax.ShapeDtypeStruct((N, D), jnp.float32),
    )(qkv, qkv)


def _final_body(o_ref, st_ref, ow, ob, gw_s, gw_a, gb1, gw2, gb2, pw, pb,
                out_ref):
    st = st_ref[...]
    agg = _dot(o_ref[...], ow[...]) + ob[...]
    g1 = jnp.maximum(_dot(st, gw_s[...]) + _dot(agg, gw_a[...]) + gb1[...], 0.0)
    logit = jnp.sum(g1 * gw2[...], axis=1, keepdims=True) + gb2[...]
    strength = 1.0 / (1.0 + jnp.exp(-logit))
    out_ref[...] = _dot(agg * strength, pw[...]) + pb[...] + st


def _final(o, states, p):
    grid = (N // _R_PRE,)
    full = lambda a: pl.BlockSpec(a.shape, lambda i: (0,) * a.ndim)
    args = [p['mha_out_W'], p['mha_out_b'].reshape(1, -1),
            p['gate_W1'][:D], p['gate_W1'][D:], p['gate_b1'].reshape(1, -1),
            p['gate_W2'].reshape(1, -1), p['gate_b2'].reshape(1, 1),
            p['proj_W'], p['proj_b'].reshape(1, -1)]
    return pl.pallas_call(
        _final_body,
        grid=grid,
        in_specs=[pl.BlockSpec((_R_PRE, D), lambda i: (i, 0)),
                  pl.BlockSpec((_R_PRE, D), lambda i: (i, 0))] +
                 [full(a) for a in args],
        out_specs=pl.BlockSpec((_R_PRE, D), lambda i: (i, 0)),
        out_shape=jax.ShapeDtypeStruct((N, D), jnp.float32),
    )(o, states, *args)


def _att_mat(gp):
    head_of_col = jnp.arange(D)[:, None] // C == jnp.arange(H)[None, :]
    a_src = jnp.where(head_of_col, gp['att_src'].reshape(-1)[:, None], 0.0)
    a_dst = jnp.where(head_of_col, gp['att_dst'].reshape(-1)[:, None], 0.0)
    return jnp.concatenate([a_src, a_dst], axis=1).astype(jnp.float32)


def _tc_pipeline(agent_states, params, cnt):
    p = params
    role_full = jnp.tile(p['role_emb'], (1, 4))
    x = None
    for l in range(2):
        gp = p['gat'][l]
        if l == 0:
            xs = [agent_states, p['agent_emb'], role_full]
            xp, a_rows, a_t, m, x0 = _gat_pre(xs, gp['W'], _att_mat(gp))
            resid = x0
        else:
            xp, a_rows, a_t, m = _gat_pre([x], gp['W'], _att_mat(gp))
            resid = x
        x = _gat_dense(cnt, xp, a_rows, a_t, m, resid,
                       gp['bias'].reshape(1, -1), p['ln_g'][l].reshape(1, -1),
                       p['ln_b'][l].reshape(1, -1))
    qkv = _mlp(x, p)
    o = _mha(qkv)
    return _final(o, agent_states, p)


def kernel(agent_states, edge_index, params):
    cnt = _build_cnt(edge_index)
    return _tc_pipeline(agent_states, params, cnt)


# speedup vs baseline: 108.6166x; 4.1872x over previous
import functools

import jax
import jax.numpy as jnp
from jax import lax
from jax.experimental import pallas as pl
from jax.experimental.pallas import tpu as pltpu
from jax.experimental.pallas import tpu_sc as plsc

N = 4096
E = 262144
D = 128
H = 4
C = D // H


_ROWS = 16
_ROUNDS = 8
_CORE_ROWS = 2048
_STRIPS = 128
_ACC = _ROWS * N
_SLICE = E // 16
_BCAP = _SLICE + _STRIPS * 7 + 64
_EXCH = E + 16 * _STRIPS * 63 + 512
_UNROLL = 8


def _cnt_body(src_hbm, dst_hbm, zeros_hbm, neg1_hbm, out_hbm,
              d_v, s_v, pk_v, bkt_v, cnt_v, offs0_v, offs_v, raw_v, go_v,
              acc_v, exch_sh, tbl_sh):
    cid = lax.axis_index("c")
    sid = lax.axis_index("s")
    core_base = cid * _CORE_ROWS
    ebase = cid * _EXCH
    lanes = lax.broadcasted_iota(jnp.int32, (16,), 0)

    def _extract(ref, idx):
        v = ref[pl.ds((idx >> 4) * 16, 16)]
        return jnp.sum(jnp.where(lanes == (idx & 15), v, 0))

    pltpu.sync_copy(neg1_hbm, bkt_v)
    for k in range(_STRIPS // 16):
        cnt_v[pl.ds(k * 16, 16)] = jnp.zeros((16,), jnp.int32)
    base_e = sid * _SLICE
    for ch in range(4):
        off = base_e + ch * 4096
        pltpu.sync_copy(dst_hbm.at[pl.ds(off, 4096)], d_v)
        pltpu.sync_copy(src_hbm.at[pl.ds(off, 4096)], s_v)

        def p1(i, c):
            for u in range(_UNROLL):
                o = (i * _UNROLL + u) * 16
                d = d_v[pl.ds(o, 16)]
                s = s_v[pl.ds(o, 16)]
                pk_v[pl.ds(ch * 4096 + o, 16)] = d * N + s
                dr = d - core_base
                valid = (dr >= 0) & (dr < _CORE_ROWS)
                b = jnp.right_shift(dr, 4)
                cnts, last = plsc.scan_count(b, valid)
                plsc.addupdate_scatter(cnt_v, [b], cnts, mask=last)
            return c

        lax.fori_loop(0, 4096 // 16 // _UNROLL, p1, 0)
    carry = jnp.int32(0)
    for k in range(_STRIPS // 16):
        c16 = cnt_v[pl.ds(k * 16, 16)]
        lp = jnp.bitwise_and(c16 + 7, -8)
        csum = plsc.cumsum(lp)
        excl = csum - lp + carry
        offs0_v[pl.ds(k * 16, 16)] = excl
        offs_v[pl.ds(k * 16, 16)] = excl
        carry = carry + jnp.sum(lp)

    def p2(i, c):
        for u in range(_UNROLL):
            o = (i * _UNROLL + u) * 16
            p = pk_v[pl.ds(o, 16)]
            dr = jnp.right_shift(p, 12) - core_base
            valid = (dr >= 0) & (dr < _CORE_ROWS)
            b = jnp.right_shift(dr, 4)
            cnts, last = plsc.scan_count(b, valid)
            g = plsc.load_gather(offs_v, [b], mask=valid)
            plsc.store_scatter(bkt_v, [g + cnts - 1], p, mask=valid)
            plsc.addupdate_scatter(offs_v, [b], cnts, mask=last)
        return c

    lax.fori_loop(0, _SLICE // 16 // _UNROLL, p2, 0)
    pltpu.sync_copy(cnt_v, tbl_sh.at[pl.ds(sid * _STRIPS, _STRIPS)])
    plsc.subcore_barrier()
    pltpu.sync_copy(tbl_sh, raw_v)

    def lay(k, carry2):
        cc = plsc.load_gather(raw_v, [lanes * _STRIPS + k])
        pc = jnp.bitwise_and(cc + 63, -64)
        csum = plsc.cumsum(pc)
        go_v[pl.ds(k * 16, 16)] = csum - pc + carry2
        return carry2 + jnp.sum(pc)

    tot = lax.fori_loop(0, _STRIPS, lay, jnp.int32(0))
    go_v[pl.ds(16 * _STRIPS, 16)] = jnp.zeros((16,), jnp.int32) + tot

    def pub(b, c):
        cb = _extract(cnt_v, b)
        lo = pl.multiple_of(_extract(offs0_v, b), 8)
        gs = pl.multiple_of(_extract(go_v, b * 16 + sid), 64)
        trip = jnp.right_shift(cb + 63, 6)

        def pchunk(j, c2):
            pltpu.sync_copy(bkt_v.at[pl.ds(lo + j * 64, 64)],
                            exch_sh.at[pl.ds(ebase + gs + j * 64, 64)])
            return c2

        lax.fori_loop(0, trip, pchunk, 0)
        return c

    lax.fori_loop(0, _STRIPS, pub, 0)

    @pl.when(sid == 0)
    def _():
        pltpu.sync_copy(neg1_hbm.at[pl.ds(0, 512)], s_v.at[pl.ds(0, 512)])
        pltpu.sync_copy(s_v.at[pl.ds(0, 512)],
                        exch_sh.at[pl.ds(ebase + pl.multiple_of(tot, 64),
                                         512)])

    plsc.subcore_barrier()
    for r in range(_ROUNDS):
        b = sid * _ROUNDS + r
        base_flat = (core_base + sid * _STRIPS + r * _ROWS) * N
        start = pl.multiple_of(_extract(go_v, b * 16), 64)
        end = _extract(go_v, (b + 1) * 16)
        trip = jnp.right_shift(end - start + 511, 9)
        pltpu.sync_copy(zeros_hbm, acc_v)

        def cchunk(j, c2):
            pltpu.sync_copy(exch_sh.at[pl.ds(ebase + start + j * 512, 512)],
                            d_v.at[pl.ds(0, 512)])

            def cb_(i, c3):
                for u in range(_UNROLL):
                    o = (i * _UNROLL + u) * 16
                    t = d_v[pl.ds(o, 16)] - base_flat
                    m = (t >= 0) & (t < _ACC)
                    cnts, last = plsc.scan_count(t, m)
                    plsc.addupdate_scatter(acc_v, [t],
                                           cnts.astype(jnp.float32), mask=last)
                return c3

            lax.fori_loop(0, 512 // 16 // _UNROLL, cb_, 0)
            return c2

        lax.fori_loop(0, trip, cchunk, 0)
        pltpu.sync_copy(acc_v, out_hbm.at[pl.ds(base_flat, _ACC)])


def _build_cnt(edge_index):
    src = edge_index[0]
    dst = edge_index[1]
    zeros = jnp.zeros((_ACC,), jnp.float32)
    neg1 = jnp.full((_BCAP,), -1, jnp.int32)
    mesh = plsc.VectorSubcoreMesh(core_axis_name="c", subcore_axis_name="s")
    k = pl.kernel(
        _cnt_body,
        out_type=jax.ShapeDtypeStruct((N * N,), jnp.float32),
        mesh=mesh,
        compiler_params=pltpu.CompilerParams(needs_layout_passes=False),
        scratch_types=[
            pltpu.VMEM((4096,), jnp.int32),
            pltpu.VMEM((4096,), jnp.int32),
            pltpu.VMEM((_SLICE,), jnp.int32),
            pltpu.VMEM((_BCAP,), jnp.int32),
            pltpu.VMEM((_STRIPS,), jnp.int32),
            pltpu.VMEM((_STRIPS,), jnp.int32),
            pltpu.VMEM((_STRIPS,), jnp.int32),
            pltpu.VMEM((16 * _STRIPS,), jnp.int32),
            pltpu.VMEM((16 * _STRIPS + 16,), jnp.int32),
            pltpu.VMEM((_ACC,), jnp.float32),
            pltpu.HBM((2 * _EXCH,), jnp.int32),
            pltpu.VMEM_SHARED((16 * _STRIPS,), jnp.int32),
        ],
    )
    return k(src, dst, zeros, neg1).reshape(N, N)


_R_PRE = 512
_R_GAT = 256
_R_MHA = 256


def _dot(a, b):
    return jnp.dot(a, b, preferred_element_type=jnp.float32)


def _pre_body(n_add, *refs):
    i = pl.program_id(0)
    x_refs = refs[:n_add]
    w_ref, acat_ref = refs[n_add], refs[n_add + 1]
    if n_add > 1:
        xp_ref, a_ref, at_ref, m_ref, x_ref = refs[n_add + 2:]
    else:
        xp_ref, a_ref, at_ref, m_ref = refs[n_add + 2:]
    x = x_refs[0][...]
    for r in x_refs[1:]:
        x = x + r[...]
    if n_add > 1:
        x_ref[...] = x
    xp = _dot(x, w_ref[...])
    a = _dot(xp, acat_ref[...])
    xp_ref[...] = xp
    a_ref[...] = a
    at_ref[...] = a.T
    blk_max = jnp.max(a, axis=0, keepdims=True)

    @pl.when(i == 0)
    def _():
        m_ref[...] = blk_max

    @pl.when(i != 0)
    def _():
        m_ref[...] = jnp.maximum(m_ref[...], blk_max)


def _gat_pre(xs, w, a_cat):
    n_add = len(xs)
    grid = (N // _R_PRE,)
    row_spec = pl.BlockSpec((_R_PRE, D), lambda i: (i, 0))
    out_specs = [
        pl.BlockSpec((_R_PRE, D), lambda i: (i, 0)),
        pl.BlockSpec((_R_PRE, 8), lambda i: (i, 0)),
        pl.BlockSpec((8, _R_PRE), lambda i: (0, i)),
        pl.BlockSpec((1, 8), lambda i: (0, 0)),
    ]
    out_shape = [
        jax.ShapeDtypeStruct((N, D), jnp.float32),
        jax.ShapeDtypeStruct((N, 8), jnp.float32),
        jax.ShapeDtypeStruct((8, N), jnp.float32),
        jax.ShapeDtypeStruct((1, 8), jnp.float32),
    ]
    if n_add > 1:
        out_specs.append(pl.BlockSpec((_R_PRE, D), lambda i: (i, 0)))
        out_shape.append(jax.ShapeDtypeStruct((N, D), jnp.float32))
    out = pl.pallas_call(
        functools.partial(_pre_body, n_add),
        grid=grid,
        in_specs=[row_spec] * n_add + [
            pl.BlockSpec((D, D), lambda i: (0, 0)),
            pl.BlockSpec((D, 8), lambda i: (0, 0)),
        ],
        out_specs=out_specs,
        out_shape=out_shape,
    )(*xs, w, a_cat)
    return out


def _lrelu(t):
    return jnp.where(t > 0, t, 0.2 * t)


def _gat_body(cnt_ref, xp_ref, a_ref, at_ref, m_ref, res_ref,
              bias_ref, g_ref, b_ref, out_ref):
    i = pl.program_id(0)
    ad = a_ref[...][:, 4:8]
    m_row = m_ref[...][:, 0:4]
    ast = at_ref[...][0:4, :]
    m_col = jnp.max(ast, axis=1, keepdims=True)
    shift = _lrelu(m_row + ad)
    ed_a = jnp.exp(ad + m_row - shift)
    ed_b = jnp.exp(0.2 * (ad + m_row) - shift)
    es_a = jnp.exp(ast - m_col)
    es_b = jnp.exp(0.2 * (ast - m_col))

    rows = lax.broadcasted_iota(jnp.int32, (_R_GAT, N), 0) + i * _R_GAT
    cols = lax.broadcasted_iota(jnp.int32, (_R_GAT, N), 1)
    cnt = cnt_ref[...] + jnp.where(rows == cols, 1.0, 0.0)

    outs = []
    for h in range(H):
        z = ast[h:h + 1, :] + ad[:, h:h + 1]
        w = cnt * jnp.where(
            z > 0,
            ed_a[:, h:h + 1] * es_a[h:h + 1, :],
            ed_b[:, h:h + 1] * es_b[h:h + 1, :],
        )
        den = jnp.sum(w, axis=1, keepdims=True)
        num = _dot(w, xp_ref[...][:, h * C:(h + 1) * C])
        outs.append(num / (den + 1e-16))
    out = jnp.concatenate(outs, axis=1) + bias_ref[...]
    mu = jnp.mean(out, axis=1, keepdims=True)
    var = jnp.mean((out - mu) ** 2, axis=1, keepdims=True)
    out = (out - mu) * lax.rsqrt(var + 1e-5) * g_ref[...] + b_ref[...]
    out_ref[...] = out + res_ref[...]


def _gat_dense(cnt, xp, a_rows, a_t, m, resid, bias, ln_g, ln_b):
    grid = (N // _R_GAT,)
    return pl.pallas_call(
        _gat_body,
        grid=grid,
        in_specs=[
            pl.BlockSpec((_R_GAT, N), lambda i: (i, 0)),
            pl.BlockSpec((N, D), lambda i: (0, 0)),
            pl.BlockSpec((_R_GAT, 8), lambda i: (i, 0)),
            pl.BlockSpec((8, N), lambda i: (0, 0)),
            pl.BlockSpec((1, 8), lambda i: (0, 0)),
            pl.BlockSpec((_R_GAT, D), lambda i: (i, 0)),
            pl.BlockSpec((1, D), lambda i: (0, 0)),
            pl.BlockSpec((1, D), lambda i: (0, 0)),
            pl.BlockSpec((1, D), lambda i: (0, 0)),
        ],
        out_specs=pl.BlockSpec((_R_GAT, D), lambda i: (i, 0)),
        out_shape=jax.ShapeDtypeStruct((N, D), jnp.float32),
    )(cnt, xp, a_rows, a_t, m, resid, bias, ln_g, ln_b)


def _mlp_body(x_ref, ew1, eb1, ew2, eb2, dw1, db1, dw2, db2, qw, qb, qkv_ref):
    x = x_ref[...]
    h1 = jnp.maximum(_dot(x, ew1[...]) + eb1[...], 0.0)
    msg = _dot(h1, ew2[...]) + eb2[...]
    d1 = jnp.maximum(_dot(msg, dw1[...]) + db1[...], 0.0)
    dec = _dot(d1, dw2[...]) + db2[...]
    qkv_ref[...] = _dot(dec, qw[...]) + qb[...]


def _mlp(x, p):
    grid = (N // _R_PRE,)
    full = lambda a: pl.BlockSpec(a.shape, lambda i: (0,) * a.ndim)
    args = [p['enc_W1'], p['enc_b1'].reshape(1, -1), p['enc_W2'],
            p['enc_b2'].reshape(1, -1), p['dec_W1'], p['dec_b1'].reshape(1, -1),
            p['dec_W2'], p['dec_b2'].reshape(1, -1), p['mha_in_W'],
            p['mha_in_b'].reshape(1, -1)]
    return pl.pallas_call(
        _mlp_body,
        grid=grid,
        in_specs=[pl.BlockSpec((_R_PRE, D), lambda i: (i, 0))] +
                 [full(a) for a in args],
        out_specs=pl.BlockSpec((_R_PRE, 3 * D), lambda i: (i, 0)),
        out_shape=jax.ShapeDtypeStruct((N, 3 * D), jnp.float32),
    )(x, *args)


def _mha_body(qt_ref, kv_ref, o_ref):
    outs = []
    for h in range(H):
        q = qt_ref[...][:, h * C:(h + 1) * C]
        k = kv_ref[...][:, D + h * C:D + (h + 1) * C]
        v = kv_ref[...][:, 2 * D + h * C:2 * D + (h + 1) * C]
        scores = lax.dot_general(q, k, (((1,), (1,)), ((), ())),
                                 preferred_element_type=jnp.float32)
        scores = scores * (1.0 / jnp.sqrt(float(C)))
        mx = jnp.max(scores, axis=1, keepdims=True)
        p = jnp.exp(scores - mx)
        s = jnp.sum(p, axis=1, keepdims=True)
        outs.append(_dot(p / s, v))
    o_ref[...] = jnp.concatenate(outs, axis=1)


def _mha(qkv):
    grid = (N // _R_MHA,)
    return pl.pallas_call(
        _mha_body,
        grid=grid,
        in_specs=[
            pl.BlockSpec((_R_MHA, 3 * D), lambda i: (i, 0)),
            pl.BlockSpec((N, 3 * D), lambda i: (0, 0)),
        ],
        out_specs=pl.BlockSpec((_R_MHA, D), lambda i: (i, 0)),
        out_shape=jax.ShapeDtypeStruct((N, D), jnp.float32),
    )(qkv, qkv)


def _final_body(o_ref, st_ref, ow, ob, gw_s, gw_a, gb1, gw2, gb2, pw, pb,
                out_ref):
    st = st_ref[...]
    agg = _dot(o_ref[...], ow[...]) + ob[...]
    g1 = jnp.maximum(_dot(st, gw_s[...]) + _dot(agg, gw_a[...]) + gb1[...], 0.0)
    logit = jnp.sum(g1 * gw2[...], axis=1, keepdims=True) + gb2[...]
    strength = 1.0 / (1.0 + jnp.exp(-logit))
    out_ref[...] = _dot(agg * strength, pw[...]) + pb[...] + st


def _final(o, states, p):
    grid = (N // _R_PRE,)
    full = lambda a: pl.BlockSpec(a.shape, lambda i: (0,) * a.ndim)
    args = [p['mha_out_W'], p['mha_out_b'].reshape(1, -1),
            p['gate_W1'][:D], p['gate_W1'][D:], p['gate_b1'].reshape(1, -1),
            p['gate_W2'].reshape(1, -1), p['gate_b2'].reshape(1, 1),
            p['proj_W'], p['proj_b'].reshape(1, -1)]
    return pl.pallas_call(
        _final_body,
        grid=grid,
        in_specs=[pl.BlockSpec((_R_PRE, D), lambda i: (i, 0)),
                  pl.BlockSpec((_R_PRE, D), lambda i: (i, 0))] +
                 [full(a) for a in args],
        out_specs=pl.BlockSpec((_R_PRE, D), lambda i: (i, 0)),
        out_shape=jax.ShapeDtypeStruct((N, D), jnp.float32),
    )(o, states, *args)


def _att_mat(gp):
    head_of_col = jnp.arange(D)[:, None] // C == jnp.arange(H)[None, :]
    a_src = jnp.where(head_of_col, gp['att_src'].reshape(-1)[:, None], 0.0)
    a_dst = jnp.where(head_of_col, gp['att_dst'].reshape(-1)[:, None], 0.0)
    return jnp.concatenate([a_src, a_dst], axis=1).astype(jnp.float32)


def _tc_pipeline(agent_states, params, cnt):
    p = params
    role_full = jnp.tile(p['role_emb'], (1, 4))
    x = None
    for l in range(2):
        gp = p['gat'][l]
        if l == 0:
            xs = [agent_states, p['agent_emb'], role_full]
            xp, a_rows, a_t, m, x0 = _gat_pre(xs, gp['W'], _att_mat(gp))
            resid = x0
        else:
            xp, a_rows, a_t, m = _gat_pre([x], gp['W'], _att_mat(gp))
            resid = x
        x = _gat_dense(cnt, xp, a_rows, a_t, m, resid,
                       gp['bias'].reshape(1, -1), p['ln_g'][l].reshape(1, -1),
                       p['ln_b'][l].reshape(1, -1))
    qkv = _mlp(x, p)
    o = _mha(qkv)
    return _final(o, agent_states, p)


def kernel(agent_states, edge_index, params):
    cnt = _build_cnt(edge_index)
    return _tc_pipeline(agent_states, params, cnt)


# speedup vs baseline: 119.4224x; 1.0995x over previous
import functools

import jax
import jax.numpy as jnp
from jax import lax
from jax.experimental import pallas as pl
from jax.experimental.pallas import tpu as pltpu
from jax.experimental.pallas import tpu_sc as plsc

N = 4096
E = 262144
D = 128
H = 4
C = D // H


_ROWS = 16
_ROUNDS = 8
_CORE_ROWS = 2048
_STRIPS = 128
_ACC = _ROWS * N
_SLICE = E // 16
_BCAP = _SLICE + _STRIPS * 7 + 64
_EXCH = E + 16 * _STRIPS * 63 + 512
_UNROLL = 8


def _cnt_body(src_hbm, dst_hbm, zeros_hbm, neg1_hbm, out_hbm,
              d_v, s_v, pk_v, bkt_v, cnt_v, offs0_v, offs_v, raw_v, go_v,
              acc_v, exch_sh, tbl_sh):
    cid = lax.axis_index("c")
    sid = lax.axis_index("s")
    core_base = cid * _CORE_ROWS
    ebase = cid * _EXCH
    lanes = lax.broadcasted_iota(jnp.int32, (16,), 0)

    def _extract(ref, idx):
        v = ref[pl.ds((idx >> 4) * 16, 16)]
        return jnp.sum(jnp.where(lanes == (idx & 15), v, 0))

    pltpu.sync_copy(neg1_hbm, bkt_v)
    for k in range(_STRIPS // 16):
        cnt_v[pl.ds(k * 16, 16)] = jnp.zeros((16,), jnp.int32)
    base_e = sid * _SLICE
    for ch in range(4):
        off = base_e + ch * 4096
        pltpu.sync_copy(dst_hbm.at[pl.ds(off, 4096)], d_v)
        pltpu.sync_copy(src_hbm.at[pl.ds(off, 4096)], s_v)

        def p1(i, c):
            for u in range(_UNROLL):
                o = (i * _UNROLL + u) * 16
                d = d_v[pl.ds(o, 16)]
                s = s_v[pl.ds(o, 16)]
                pk_v[pl.ds(ch * 4096 + o, 16)] = d * N + s
                dr = d - core_base
                valid = (dr >= 0) & (dr < _CORE_ROWS)
                b = jnp.right_shift(dr, 4)
                cnts, last = plsc.scan_count(b, valid)
                plsc.addupdate_scatter(cnt_v, [b], cnts, mask=last)
            return c

        lax.fori_loop(0, 4096 // 16 // _UNROLL, p1, 0)
    carry = jnp.int32(0)
    for k in range(_STRIPS // 16):
        c16 = cnt_v[pl.ds(k * 16, 16)]
        lp = jnp.bitwise_and(c16 + 7, -8)
        csum = plsc.cumsum(lp)
        excl = csum - lp + carry
        offs0_v[pl.ds(k * 16, 16)] = excl
        offs_v[pl.ds(k * 16, 16)] = excl
        carry = carry + jnp.sum(lp)

    def p2(i, c):
        for u in range(_UNROLL):
            o = (i * _UNROLL + u) * 16
            p = pk_v[pl.ds(o, 16)]
            dr = jnp.right_shift(p, 12) - core_base
            valid = (dr >= 0) & (dr < _CORE_ROWS)
            b = jnp.right_shift(dr, 4)
            cnts, last = plsc.scan_count(b, valid)
            g = plsc.load_gather(offs_v, [b], mask=valid)
            plsc.store_scatter(bkt_v, [g + cnts - 1], p, mask=valid)
            plsc.addupdate_scatter(offs_v, [b], cnts, mask=last)
        return c

    lax.fori_loop(0, _SLICE // 16 // _UNROLL, p2, 0)
    pltpu.sync_copy(cnt_v, tbl_sh.at[pl.ds(sid * _STRIPS, _STRIPS)])
    plsc.subcore_barrier()
    pltpu.sync_copy(tbl_sh, raw_v)

    def lay(k, carry2):
        cc = plsc.load_gather(raw_v, [lanes * _STRIPS + k])
        pc = jnp.bitwise_and(cc + 63, -64)
        csum = plsc.cumsum(pc)
        go_v[pl.ds(k * 16, 16)] = csum - pc + carry2
        return carry2 + jnp.sum(pc)

    tot = lax.fori_loop(0, _STRIPS, lay, jnp.int32(0))
    go_v[pl.ds(16 * _STRIPS, 16)] = jnp.zeros((16,), jnp.int32) + tot

    def pub(b, c):
        cb = _extract(cnt_v, b)
        lo = pl.multiple_of(_extract(offs0_v, b), 8)
        gs = pl.multiple_of(_extract(go_v, b * 16 + sid), 64)
        trip = jnp.right_shift(cb + 63, 6)

        def pchunk(j, c2):
            pltpu.sync_copy(bkt_v.at[pl.ds(lo + j * 64, 64)],
                            exch_sh.at[pl.ds(ebase + gs + j * 64, 64)])
            return c2

        lax.fori_loop(0, trip, pchunk, 0)
        return c

    lax.fori_loop(0, _STRIPS, pub, 0)

    @pl.when(sid == 0)
    def _():
        pltpu.sync_copy(neg1_hbm.at[pl.ds(0, 512)], s_v.at[pl.ds(0, 512)])
        pltpu.sync_copy(s_v.at[pl.ds(0, 512)],
                        exch_sh.at[pl.ds(ebase + pl.multiple_of(tot, 64),
                                         512)])

    plsc.subcore_barrier()
    for r in range(_ROUNDS):
        b = sid * _ROUNDS + r
        base_flat = (core_base + sid * _STRIPS + r * _ROWS) * N
        start = pl.multiple_of(_extract(go_v, b * 16), 64)
        end = _extract(go_v, (b + 1) * 16)
        trip = jnp.right_shift(end - start + 511, 9)
        pltpu.sync_copy(zeros_hbm, acc_v)

        def cchunk(j, c2):
            pltpu.sync_copy(exch_sh.at[pl.ds(ebase + start + j * 512, 512)],
                            d_v.at[pl.ds(0, 512)])

            def cb_(i, c3):
                for u in range(_UNROLL):
                    o = (i * _UNROLL + u) * 16
                    t = d_v[pl.ds(o, 16)] - base_flat
                    m = (t >= 0) & (t < _ACC)
                    cnts, last = plsc.scan_count(t, m)
                    plsc.addupdate_scatter(acc_v, [t],
                                           cnts.astype(jnp.float32), mask=last)
                return c3

            lax.fori_loop(0, 512 // 16 // _UNROLL, cb_, 0)
            return c2

        lax.fori_loop(0, trip, cchunk, 0)
        pltpu.sync_copy(acc_v, out_hbm.at[pl.ds(base_flat, _ACC)])


def _build_cnt(edge_index):
    src = edge_index[0]
    dst = edge_index[1]
    zeros = jnp.zeros((_ACC,), jnp.float32)
    neg1 = jnp.full((_BCAP,), -1, jnp.int32)
    mesh = plsc.VectorSubcoreMesh(core_axis_name="c", subcore_axis_name="s")
    k = pl.kernel(
        _cnt_body,
        out_type=jax.ShapeDtypeStruct((N * N,), jnp.float32),
        mesh=mesh,
        compiler_params=pltpu.CompilerParams(needs_layout_passes=False),
        scratch_types=[
            pltpu.VMEM((4096,), jnp.int32),
            pltpu.VMEM((4096,), jnp.int32),
            pltpu.VMEM((_SLICE,), jnp.int32),
            pltpu.VMEM((_BCAP,), jnp.int32),
            pltpu.VMEM((_STRIPS,), jnp.int32),
            pltpu.VMEM((_STRIPS,), jnp.int32),
            pltpu.VMEM((_STRIPS,), jnp.int32),
            pltpu.VMEM((16 * _STRIPS,), jnp.int32),
            pltpu.VMEM((16 * _STRIPS + 16,), jnp.int32),
            pltpu.VMEM((_ACC,), jnp.float32),
            pltpu.HBM((2 * _EXCH,), jnp.int32),
            pltpu.VMEM_SHARED((16 * _STRIPS,), jnp.int32),
        ],
    )
    return k(src, dst, zeros, neg1).reshape(N, N)


_R_PRE = 512
_R_GAT = 256
_R_MHA = 256


def _dot(a, b):
    return jnp.dot(a, b, preferred_element_type=jnp.float32)


def _pre_body(n_add, *refs):
    i = pl.program_id(0)
    x_refs = refs[:n_add]
    w_ref, acat_ref = refs[n_add], refs[n_add + 1]
    if n_add > 1:
        xp_ref, a_ref, at_ref, m_ref, x_ref = refs[n_add + 2:]
    else:
        xp_ref, a_ref, at_ref, m_ref = refs[n_add + 2:]
    x = x_refs[0][...]
    for r in x_refs[1:]:
        x = x + r[...]
    if n_add > 1:
        x_ref[...] = x
    xp = _dot(x, w_ref[...])
    a = _dot(xp, acat_ref[...])
    xp_ref[...] = xp.astype(jnp.bfloat16)
    a_ref[...] = a
    at_ref[...] = a.T
    blk_max = jnp.max(a, axis=0, keepdims=True)

    @pl.when(i == 0)
    def _():
        m_ref[...] = blk_max

    @pl.when(i != 0)
    def _():
        m_ref[...] = jnp.maximum(m_ref[...], blk_max)


def _gat_pre(xs, w, a_cat):
    n_add = len(xs)
    grid = (N // _R_PRE,)
    row_spec = pl.BlockSpec((_R_PRE, D), lambda i: (i, 0))
    out_specs = [
        pl.BlockSpec((_R_PRE, D), lambda i: (i, 0)),
        pl.BlockSpec((_R_PRE, 8), lambda i: (i, 0)),
        pl.BlockSpec((8, _R_PRE), lambda i: (0, i)),
        pl.BlockSpec((1, 8), lambda i: (0, 0)),
    ]
    out_shape = [
        jax.ShapeDtypeStruct((N, D), jnp.bfloat16),
        jax.ShapeDtypeStruct((N, 8), jnp.float32),
        jax.ShapeDtypeStruct((8, N), jnp.float32),
        jax.ShapeDtypeStruct((1, 8), jnp.float32),
    ]
    if n_add > 1:
        out_specs.append(pl.BlockSpec((_R_PRE, D), lambda i: (i, 0)))
        out_shape.append(jax.ShapeDtypeStruct((N, D), jnp.float32))
    out = pl.pallas_call(
        functools.partial(_pre_body, n_add),
        grid=grid,
        in_specs=[row_spec] * n_add + [
            pl.BlockSpec((D, D), lambda i: (0, 0)),
            pl.BlockSpec((D, 8), lambda i: (0, 0)),
        ],
        out_specs=out_specs,
        out_shape=out_shape,
    )(*xs, w, a_cat)
    return out


def _lrelu(t):
    return jnp.where(t > 0, t, 0.2 * t)


def _gat_body(cnt_ref, xp_ref, xpt_ref, a_ref, at_ref, m_ref, res_ref,
              bias_ref, g_ref, b_ref, out_ref):
    a_blk = a_ref[...]
    ad = a_blk[:, 4:8]
    a_self = a_blk[:, 0:4]
    m_row = m_ref[...][:, 0:4]
    ast = at_ref[...][0:4, :]
    m_col = jnp.max(ast, axis=1, keepdims=True)
    shift = _lrelu(m_row + ad)
    ed_a = jnp.exp(ad + m_row - shift)
    ed_b = jnp.exp(0.2 * (ad + m_row) - shift)
    es_a = jnp.exp(ast - m_col)
    es_b = jnp.exp(0.2 * (ast - m_col))
    wdd = jnp.exp(_lrelu(a_self + ad) - shift)

    cnt = cnt_ref[...]
    xpt = xpt_ref[...].astype(jnp.float32)
    outs = []
    for h in range(H):
        msk = ast[h:h + 1, :] > -ad[:, h:h + 1]
        s1 = jnp.where(msk, es_a[h:h + 1, :], es_b[h:h + 1, :])
        s2 = jnp.where(msk, ed_a[:, h:h + 1], ed_b[:, h:h + 1])
        w = cnt * s1 * s2
        den = jnp.sum(w, axis=1, keepdims=True) + wdd[:, h:h + 1]
        num = _dot(w.astype(jnp.bfloat16), xp_ref[...][:, h * C:(h + 1) * C])
        num = num + wdd[:, h:h + 1] * xpt[:, h * C:(h + 1) * C]
        outs.append(num / (den + 1e-16))
    out = jnp.concatenate(outs, axis=1) + bias_ref[...]
    mu = jnp.mean(out, axis=1, keepdims=True)
    var = jnp.mean((out - mu) ** 2, axis=1, keepdims=True)
    out = (out - mu) * lax.rsqrt(var + 1e-5) * g_ref[...] + b_ref[...]
    out_ref[...] = out + res_ref[...]


def _gat_dense(cnt, xp, a_rows, a_t, m, resid, bias, ln_g, ln_b):
    grid = (N // _R_GAT,)
    return pl.pallas_call(
        _gat_body,
        grid=grid,
        in_specs=[
            pl.BlockSpec((_R_GAT, N), lambda i: (i, 0)),
            pl.BlockSpec((N, D), lambda i: (0, 0)),
            pl.BlockSpec((_R_GAT, D), lambda i: (i, 0)),
            pl.BlockSpec((_R_GAT, 8), lambda i: (i, 0)),
            pl.BlockSpec((8, N), lambda i: (0, 0)),
            pl.BlockSpec((1, 8), lambda i: (0, 0)),
            pl.BlockSpec((_R_GAT, D), lambda i: (i, 0)),
            pl.BlockSpec((1, D), lambda i: (0, 0)),
            pl.BlockSpec((1, D), lambda i: (0, 0)),
            pl.BlockSpec((1, D), lambda i: (0, 0)),
        ],
        out_specs=pl.BlockSpec((_R_GAT, D), lambda i: (i, 0)),
        out_shape=jax.ShapeDtypeStruct((N, D), jnp.float32),
    )(cnt, xp, xp, a_rows, a_t, m, resid, bias, ln_g, ln_b)


def _mlp_body(x_ref, ew1, eb1, ew2, eb2, dw1, db1, dw2, db2, qw, qb, scl,
              qkv_ref):
    x = x_ref[...]
    h1 = jnp.maximum(_dot(x, ew1[...]) + eb1[...], 0.0)
    msg = _dot(h1, ew2[...]) + eb2[...]
    d1 = jnp.maximum(_dot(msg, dw1[...]) + db1[...], 0.0)
    dec = _dot(d1, dw2[...]) + db2[...]
    qkv = (_dot(dec, qw[...]) + qb[...]) * scl[...]
    qkv_ref[...] = qkv.astype(jnp.bfloat16)


def _mlp(x, p):
    grid = (N // _R_PRE,)
    full = lambda a: pl.BlockSpec(a.shape, lambda i: (0,) * a.ndim)
    scl = jnp.concatenate([jnp.full((D,), C ** -0.5, jnp.float32),
                           jnp.ones((2 * D,), jnp.float32)]).reshape(1, -1)
    args = [p['enc_W1'], p['enc_b1'].reshape(1, -1), p['enc_W2'],
            p['enc_b2'].reshape(1, -1), p['dec_W1'], p['dec_b1'].reshape(1, -1),
            p['dec_W2'], p['dec_b2'].reshape(1, -1), p['mha_in_W'],
            p['mha_in_b'].reshape(1, -1), scl]
    return pl.pallas_call(
        _mlp_body,
        grid=grid,
        in_specs=[pl.BlockSpec((_R_PRE, D), lambda i: (i, 0))] +
                 [full(a) for a in args],
        out_specs=pl.BlockSpec((_R_PRE, 3 * D), lambda i: (i, 0)),
        out_shape=jax.ShapeDtypeStruct((N, 3 * D), jnp.bfloat16),
    )(x, *args)


def _mha_body(qt_ref, kv_ref, o_ref):
    outs = []
    for h in range(H):
        q = qt_ref[...][:, h * C:(h + 1) * C]
        k = kv_ref[...][:, D + h * C:D + (h + 1) * C]
        v = kv_ref[...][:, 2 * D + h * C:2 * D + (h + 1) * C]
        scores = lax.dot_general(q, k, (((1,), (1,)), ((), ())),
                                 preferred_element_type=jnp.float32)
        p = jnp.exp(scores)
        s = jnp.sum(p, axis=1, keepdims=True)
        outs.append(_dot(p.astype(jnp.bfloat16), v) / s)
    o_ref[...] = jnp.concatenate(outs, axis=1)


def _mha(qkv):
    grid = (N // _R_MHA,)
    return pl.pallas_call(
        _mha_body,
        grid=grid,
        in_specs=[
            pl.BlockSpec((_R_MHA, 3 * D), lambda i: (i, 0)),
            pl.BlockSpec((N, 3 * D), lambda i: (0, 0)),
        ],
        out_specs=pl.BlockSpec((_R_MHA, D), lambda i: (i, 0)),
        out_shape=jax.ShapeDtypeStruct((N, D), jnp.float32),
    )(qkv, qkv)


def _final_body(o_ref, st_ref, ow, ob, gw_s, gw_a, gb1, gw2, gb2, pw, pb,
                out_ref):
    st = st_ref[...]
    agg = _dot(o_ref[...], ow[...]) + ob[...]
    g1 = jnp.maximum(_dot(st, gw_s[...]) + _dot(agg, gw_a[...]) + gb1[...], 0.0)
    logit = jnp.sum(g1 * gw2[...], axis=1, keepdims=True) + gb2[...]
    strength = 1.0 / (1.0 + jnp.exp(-logit))
    out_ref[...] = _dot(agg * strength, pw[...]) + pb[...] + st


def _final(o, states, p):
    grid = (N // _R_PRE,)
    full = lambda a: pl.BlockSpec(a.shape, lambda i: (0,) * a.ndim)
    args = [p['mha_out_W'], p['mha_out_b'].reshape(1, -1),
            p['gate_W1'][:D], p['gate_W1'][D:], p['gate_b1'].reshape(1, -1),
            p['gate_W2'].reshape(1, -1), p['gate_b2'].reshape(1, 1),
            p['proj_W'], p['proj_b'].reshape(1, -1)]
    return pl.pallas_call(
        _final_body,
        grid=grid,
        in_specs=[pl.BlockSpec((_R_PRE, D), lambda i: (i, 0)),
                  pl.BlockSpec((_R_PRE, D), lambda i: (i, 0))] +
                 [full(a) for a in args],
        out_specs=pl.BlockSpec((_R_PRE, D), lambda i: (i, 0)),
        out_shape=jax.ShapeDtypeStruct((N, D), jnp.float32),
    )(o, states, *args)


def _att_mat(gp):
    head_of_col = jnp.arange(D)[:, None] // C == jnp.arange(H)[None, :]
    a_src = jnp.where(head_of_col, gp['att_src'].reshape(-1)[:, None], 0.0)
    a_dst = jnp.where(head_of_col, gp['att_dst'].reshape(-1)[:, None], 0.0)
    return jnp.concatenate([a_src, a_dst], axis=1).astype(jnp.float32)


def _tc_pipeline(agent_states, params, cnt):
    p = params
    role_full = jnp.tile(p['role_emb'], (1, 4))
    x = None
    for l in range(2):
        gp = p['gat'][l]
        if l == 0:
            xs = [agent_states, p['agent_emb'], role_full]
            xp, a_rows, a_t, m, x0 = _gat_pre(xs, gp['W'], _att_mat(gp))
            resid = x0
        else:
            xp, a_rows, a_t, m = _gat_pre([x], gp['W'], _att_mat(gp))
            resid = x
        x = _gat_dense(cnt, xp, a_rows, a_t, m, resid,
                       gp['bias'].reshape(1, -1), p['ln_g'][l].reshape(1, -1),
                       p['ln_b'][l].reshape(1, -1))
    qkv = _mlp(x, p)
    o = _mha(qkv)
    return _final(o, agent_states, p)


def kernel(agent_states, edge_index, params):
    cnt = _build_cnt(edge_index)
    return _tc_pipeline(agent_states, params, cnt)


# speedup vs baseline: 148.8610x; 1.2465x over previous
import functools

import jax
import jax.numpy as jnp
from jax import lax
from jax.experimental import pallas as pl
from jax.experimental.pallas import tpu as pltpu
from jax.experimental.pallas import tpu_sc as plsc

N = 4096
E = 262144
D = 128
H = 4
C = D // H


_ROWS = 16
_ROUNDS = 8
_CORE_ROWS = 2048
_STRIPS = 128
_ACC = _ROWS * N
_SLICE = E // 16
_BCAP = _SLICE + _STRIPS * 7 + 64
_EXCH = E + 16 * _STRIPS * 63 + 512
_UNROLL = 8


def _cnt_body(src_hbm, dst_hbm, zeros_hbm, neg1_hbm, out_hbm,
              d_v, s_v, pk_v, bkt_v, cnt_v, offs0_v, offs_v, raw_v, go_v,
              acc_v, exch_sh, tbl_sh):
    cid = lax.axis_index("c")
    sid = lax.axis_index("s")
    core_base = cid * _CORE_ROWS
    ebase = cid * _EXCH
    lanes = lax.broadcasted_iota(jnp.int32, (16,), 0)

    def _extract(ref, idx):
        v = ref[pl.ds((idx >> 4) * 16, 16)]
        return jnp.sum(jnp.where(lanes == (idx & 15), v, 0))

    pltpu.sync_copy(neg1_hbm, bkt_v)
    for k in range(_STRIPS // 16):
        cnt_v[pl.ds(k * 16, 16)] = jnp.zeros((16,), jnp.int32)
    base_e = sid * _SLICE
    for ch in range(4):
        off = base_e + ch * 4096
        pltpu.sync_copy(dst_hbm.at[pl.ds(off, 4096)], d_v)
        pltpu.sync_copy(src_hbm.at[pl.ds(off, 4096)], s_v)

        def p1(i, c):
            for u in range(_UNROLL):
                o = (i * _UNROLL + u) * 16
                d = d_v[pl.ds(o, 16)]
                s = s_v[pl.ds(o, 16)]
                pk_v[pl.ds(ch * 4096 + o, 16)] = d * N + s
                dr = d - core_base
                valid = (dr >= 0) & (dr < _CORE_ROWS)
                b = jnp.right_shift(dr, 4)
                cnts, last = plsc.scan_count(b, valid)
                plsc.addupdate_scatter(cnt_v, [b], cnts, mask=last)
            return c

        lax.fori_loop(0, 4096 // 16 // _UNROLL, p1, 0)
    carry = jnp.int32(0)
    for k in range(_STRIPS // 16):
        c16 = cnt_v[pl.ds(k * 16, 16)]
        lp = jnp.bitwise_and(c16 + 7, -8)
        csum = plsc.cumsum(lp)
        excl = csum - lp + carry
        offs0_v[pl.ds(k * 16, 16)] = excl
        offs_v[pl.ds(k * 16, 16)] = excl
        carry = carry + jnp.sum(lp)

    def p2(i, c):
        for u in range(_UNROLL):
            o = (i * _UNROLL + u) * 16
            p = pk_v[pl.ds(o, 16)]
            dr = jnp.right_shift(p, 12) - core_base
            valid = (dr >= 0) & (dr < _CORE_ROWS)
            b = jnp.right_shift(dr, 4)
            cnts, last = plsc.scan_count(b, valid)
            g = plsc.load_gather(offs_v, [b], mask=valid)
            plsc.store_scatter(bkt_v, [g + cnts - 1], p, mask=valid)
            plsc.addupdate_scatter(offs_v, [b], cnts, mask=last)
        return c

    lax.fori_loop(0, _SLICE // 16 // _UNROLL, p2, 0)
    pltpu.sync_copy(cnt_v, tbl_sh.at[pl.ds(sid * _STRIPS, _STRIPS)])
    plsc.subcore_barrier()
    pltpu.sync_copy(tbl_sh, raw_v)

    def lay(k, carry2):
        cc = plsc.load_gather(raw_v, [lanes * _STRIPS + k])
        pc = jnp.bitwise_and(cc + 63, -64)
        csum = plsc.cumsum(pc)
        go_v[pl.ds(k * 16, 16)] = csum - pc + carry2
        return carry2 + jnp.sum(pc)

    tot = lax.fori_loop(0, _STRIPS, lay, jnp.int32(0))
    go_v[pl.ds(16 * _STRIPS, 16)] = jnp.zeros((16,), jnp.int32) + tot

    def pub(b, c):
        cb = _extract(cnt_v, b)
        lo = pl.multiple_of(_extract(offs0_v, b), 8)
        gs = pl.multiple_of(_extract(go_v, b * 16 + sid), 64)
        trip = jnp.right_shift(cb + 63, 6)

        def pchunk(j, c2):
            pltpu.sync_copy(bkt_v.at[pl.ds(lo + j * 64, 64)],
                            exch_sh.at[pl.ds(ebase + gs + j * 64, 64)])
            return c2

        lax.fori_loop(0, trip, pchunk, 0)
        return c

    lax.fori_loop(0, _STRIPS, pub, 0)

    @pl.when(sid == 0)
    def _():
        pltpu.sync_copy(neg1_hbm.at[pl.ds(0, 512)], s_v.at[pl.ds(0, 512)])
        pltpu.sync_copy(s_v.at[pl.ds(0, 512)],
                        exch_sh.at[pl.ds(ebase + pl.multiple_of(tot, 64),
                                         512)])

    plsc.subcore_barrier()
    for r in range(_ROUNDS):
        b = sid * _ROUNDS + r
        base_flat = (core_base + sid * _STRIPS + r * _ROWS) * N
        start = pl.multiple_of(_extract(go_v, b * 16), 64)
        end = _extract(go_v, (b + 1) * 16)
        trip = jnp.right_shift(end - start + 511, 9)
        pltpu.sync_copy(zeros_hbm, acc_v)

        def cchunk(j, c2):
            pltpu.sync_copy(exch_sh.at[pl.ds(ebase + start + j * 512, 512)],
                            d_v.at[pl.ds(0, 512)])

            def cb_(i, c3):
                for u in range(_UNROLL):
                    o = (i * _UNROLL + u) * 16
                    t = d_v[pl.ds(o, 16)] - base_flat
                    m = (t >= 0) & (t < _ACC)
                    cnts, last = plsc.scan_count(t, m)
                    plsc.addupdate_scatter(acc_v, [t],
                                           cnts.astype(jnp.float32), mask=last)
                return c3

            lax.fori_loop(0, 512 // 16 // _UNROLL, cb_, 0)
            return c2

        lax.fori_loop(0, trip, cchunk, 0)
        pltpu.sync_copy(acc_v, out_hbm.at[pl.ds(base_flat, _ACC)])


def _build_cnt(edge_index):
    src = edge_index[0]
    dst = edge_index[1]
    zeros = jnp.zeros((_ACC,), jnp.float32)
    neg1 = jnp.full((_BCAP,), -1, jnp.int32)
    mesh = plsc.VectorSubcoreMesh(core_axis_name="c", subcore_axis_name="s")
    k = pl.kernel(
        _cnt_body,
        out_type=jax.ShapeDtypeStruct((N * N,), jnp.float32),
        mesh=mesh,
        compiler_params=pltpu.CompilerParams(needs_layout_passes=False),
        scratch_types=[
            pltpu.VMEM((4096,), jnp.int32),
            pltpu.VMEM((4096,), jnp.int32),
            pltpu.VMEM((_SLICE,), jnp.int32),
            pltpu.VMEM((_BCAP,), jnp.int32),
            pltpu.VMEM((_STRIPS,), jnp.int32),
            pltpu.VMEM((_STRIPS,), jnp.int32),
            pltpu.VMEM((_STRIPS,), jnp.int32),
            pltpu.VMEM((16 * _STRIPS,), jnp.int32),
            pltpu.VMEM((16 * _STRIPS + 16,), jnp.int32),
            pltpu.VMEM((_ACC,), jnp.float32),
            pltpu.HBM((2 * _EXCH,), jnp.int32),
            pltpu.VMEM_SHARED((16 * _STRIPS,), jnp.int32),
        ],
    )
    return k(src, dst, zeros, neg1).reshape(N, N)


_R_PRE = 512
_R_GAT = 256
_R_MHA = 256


def _dot(a, b):
    return jnp.dot(a, b, preferred_element_type=jnp.float32)


def _pre_body(n_add, *refs):
    i = pl.program_id(0)
    x_refs = refs[:n_add]
    w_ref, acat_ref = refs[n_add], refs[n_add + 1]
    if n_add > 1:
        xp_ref, a_ref, at_ref, m_ref, x_ref = refs[n_add + 2:]
    else:
        xp_ref, a_ref, at_ref, m_ref = refs[n_add + 2:]
    x = x_refs[0][...]
    for r in x_refs[1:]:
        x = x + r[...]
    if n_add > 1:
        x_ref[...] = x
    xp = _dot(x, w_ref[...])
    a = _dot(xp, acat_ref[...])
    xpb = xp.astype(jnp.bfloat16)
    one = jnp.ones((xp.shape[0], 1), jnp.bfloat16)
    zero = jnp.zeros((xp.shape[0], 7), jnp.bfloat16)
    parts = []
    for h in range(H):
        parts += [xpb[:, h * C:(h + 1) * C], one, zero]
    xp_ref[...] = jnp.concatenate(parts, axis=1)
    a_ref[...] = a
    at_ref[...] = a.T
    blk_max = jnp.max(a, axis=0, keepdims=True)

    @pl.when(i == 0)
    def _():
        m_ref[...] = blk_max

    @pl.when(i != 0)
    def _():
        m_ref[...] = jnp.maximum(m_ref[...], blk_max)


def _gat_pre(xs, w, a_cat):
    n_add = len(xs)
    grid = (N // _R_PRE,)
    row_spec = pl.BlockSpec((_R_PRE, D), lambda i: (i, 0))
    out_specs = [
        pl.BlockSpec((_R_PRE, D + 32), lambda i: (i, 0)),
        pl.BlockSpec((_R_PRE, 8), lambda i: (i, 0)),
        pl.BlockSpec((8, _R_PRE), lambda i: (0, i)),
        pl.BlockSpec((1, 8), lambda i: (0, 0)),
    ]
    out_shape = [
        jax.ShapeDtypeStruct((N, D + 32), jnp.bfloat16),
        jax.ShapeDtypeStruct((N, 8), jnp.float32),
        jax.ShapeDtypeStruct((8, N), jnp.float32),
        jax.ShapeDtypeStruct((1, 8), jnp.float32),
    ]
    if n_add > 1:
        out_specs.append(pl.BlockSpec((_R_PRE, D), lambda i: (i, 0)))
        out_shape.append(jax.ShapeDtypeStruct((N, D), jnp.float32))
    out = pl.pallas_call(
        functools.partial(_pre_body, n_add),
        grid=grid,
        in_specs=[row_spec] * n_add + [
            pl.BlockSpec((D, D), lambda i: (0, 0)),
            pl.BlockSpec((D, 8), lambda i: (0, 0)),
        ],
        out_specs=out_specs,
        out_shape=out_shape,
    )(*xs, w, a_cat)
    return out


def _lrelu(t):
    return jnp.where(t > 0, t, 0.2 * t)


def _gat_body(cnt_ref, xp_ref, xpt_ref, a_ref, at_ref, m_ref, res_ref,
              bias_ref, g_ref, b_ref, out_ref):
    a_blk = a_ref[...]
    ad = a_blk[:, 4:8]
    a_self = a_blk[:, 0:4]
    m_row = m_ref[...][:, 0:4]
    ast = at_ref[...][0:4, :]
    m_col = jnp.max(ast, axis=1, keepdims=True)
    shift = _lrelu(m_row + ad)
    ed_a = jnp.exp(ad + m_row - shift)
    ed_b = jnp.exp(0.2 * (ad + m_row) - shift)
    es_a = jnp.exp(ast - m_col)
    es_b = jnp.exp(0.2 * (ast - m_col))
    wdd = jnp.exp(_lrelu(a_self + ad) - shift)

    cnt = cnt_ref[...].astype(jnp.bfloat16)
    ast_b = ast.astype(jnp.bfloat16)
    nad_b = (-ad).astype(jnp.bfloat16)
    esa_b = es_a.astype(jnp.bfloat16)
    esb_b = es_b.astype(jnp.bfloat16)
    eda_b = ed_a.astype(jnp.bfloat16)
    edb_b = ed_b.astype(jnp.bfloat16)
    xpe = xp_ref[...]
    outs = []
    for h in range(H):
        msk = ast_b[h:h + 1, :] > nad_b[:, h:h + 1]
        s1 = jnp.where(msk, esa_b[h:h + 1, :], esb_b[h:h + 1, :])
        s2 = jnp.where(msk, eda_b[:, h:h + 1], edb_b[:, h:h + 1])
        w = cnt * s1 * s2
        nd = _dot(w, xpe[:, h * (C + 8):h * (C + 8) + C + 8])
        wdd_h = wdd[:, h:h + 1]
        num = nd[:, :C] + wdd_h * xpt_ref[...][:, h * (C + 8):
                                               h * (C + 8) + C].astype(
                                                   jnp.float32)
        den = nd[:, C:C + 1] + wdd_h
        outs.append(num / (den + 1e-16))
    out = jnp.concatenate(outs, axis=1) + bias_ref[...]
    mu = jnp.mean(out, axis=1, keepdims=True)
    var = jnp.mean((out - mu) ** 2, axis=1, keepdims=True)
    out = (out - mu) * lax.rsqrt(var + 1e-5) * g_ref[...] + b_ref[...]
    out_ref[...] = out + res_ref[...]


def _gat_dense(cnt, xp, a_rows, a_t, m, resid, bias, ln_g, ln_b):
    grid = (N // _R_GAT,)
    return pl.pallas_call(
        _gat_body,
        grid=grid,
        in_specs=[
            pl.BlockSpec((_R_GAT, N), lambda i: (i, 0)),
            pl.BlockSpec((N, D + 32), lambda i: (0, 0)),
            pl.BlockSpec((_R_GAT, D + 32), lambda i: (i, 0)),
            pl.BlockSpec((_R_GAT, 8), lambda i: (i, 0)),
            pl.BlockSpec((8, N), lambda i: (0, 0)),
            pl.BlockSpec((1, 8), lambda i: (0, 0)),
            pl.BlockSpec((_R_GAT, D), lambda i: (i, 0)),
            pl.BlockSpec((1, D), lambda i: (0, 0)),
            pl.BlockSpec((1, D), lambda i: (0, 0)),
            pl.BlockSpec((1, D), lambda i: (0, 0)),
        ],
        out_specs=pl.BlockSpec((_R_GAT, D), lambda i: (i, 0)),
        out_shape=jax.ShapeDtypeStruct((N, D), jnp.float32),
    )(cnt, xp, xp, a_rows, a_t, m, resid, bias, ln_g, ln_b)


def _mlp_body(x_ref, ew1, eb1, ew2, eb2, dw1, db1, dw2, db2, qw, qb, scl,
              qkv_ref):
    x = x_ref[...]
    h1 = jnp.maximum(_dot(x, ew1[...]) + eb1[...], 0.0)
    msg = _dot(h1, ew2[...]) + eb2[...]
    d1 = jnp.maximum(_dot(msg, dw1[...]) + db1[...], 0.0)
    dec = _dot(d1, dw2[...]) + db2[...]
    qkv = (_dot(dec, qw[...]) + qb[...]) * scl[...]
    qkv_ref[...] = qkv.astype(jnp.bfloat16)


def _mlp(x, p):
    grid = (N // _R_PRE,)
    full = lambda a: pl.BlockSpec(a.shape, lambda i: (0,) * a.ndim)
    scl = jnp.concatenate([jnp.full((D,), C ** -0.5, jnp.float32),
                           jnp.ones((2 * D,), jnp.float32)]).reshape(1, -1)
    args = [p['enc_W1'], p['enc_b1'].reshape(1, -1), p['enc_W2'],
            p['enc_b2'].reshape(1, -1), p['dec_W1'], p['dec_b1'].reshape(1, -1),
            p['dec_W2'], p['dec_b2'].reshape(1, -1), p['mha_in_W'],
            p['mha_in_b'].reshape(1, -1), scl]
    return pl.pallas_call(
        _mlp_body,
        grid=grid,
        in_specs=[pl.BlockSpec((_R_PRE, D), lambda i: (i, 0))] +
                 [full(a) for a in args],
        out_specs=pl.BlockSpec((_R_PRE, 3 * D), lambda i: (i, 0)),
        out_shape=jax.ShapeDtypeStruct((N, 3 * D), jnp.bfloat16),
    )(x, *args)


def _mha_body(qt_ref, kv_ref, o_ref):
    outs = []
    for h in range(H):
        q = qt_ref[...][:, h * C:(h + 1) * C]
        k = kv_ref[...][:, D + h * C:D + (h + 1) * C]
        v = kv_ref[...][:, 2 * D + h * C:2 * D + (h + 1) * C]
        scores = lax.dot_general(q, k, (((1,), (1,)), ((), ())),
                                 preferred_element_type=jnp.float32)
        p = jnp.exp(scores)
        s = jnp.sum(p, axis=1, keepdims=True)
        outs.append(_dot(p.astype(jnp.bfloat16), v) / s)
    o_ref[...] = jnp.concatenate(outs, axis=1)


def _mha(qkv):
    grid = (N // _R_MHA,)
    return pl.pallas_call(
        _mha_body,
        grid=grid,
        in_specs=[
            pl.BlockSpec((_R_MHA, 3 * D), lambda i: (i, 0)),
            pl.BlockSpec((N, 3 * D), lambda i: (0, 0)),
        ],
        out_specs=pl.BlockSpec((_R_MHA, D), lambda i: (i, 0)),
        out_shape=jax.ShapeDtypeStruct((N, D), jnp.float32),
    )(qkv, qkv)


def _final_body(o_ref, st_ref, ow, ob, gw_s, gw_a, gb1, gw2, gb2, pw, pb,
                out_ref):
    st = st_ref[...]
    agg = _dot(o_ref[...], ow[...]) + ob[...]
    g1 = jnp.maximum(_dot(st, gw_s[...]) + _dot(agg, gw_a[...]) + gb1[...], 0.0)
    logit = jnp.sum(g1 * gw2[...], axis=1, keepdims=True) + gb2[...]
    strength = 1.0 / (1.0 + jnp.exp(-logit))
    out_ref[...] = _dot(agg * strength, pw[...]) + pb[...] + st


def _final(o, states, p):
    grid = (N // _R_PRE,)
    full = lambda a: pl.BlockSpec(a.shape, lambda i: (0,) * a.ndim)
    args = [p['mha_out_W'], p['mha_out_b'].reshape(1, -1),
            p['gate_W1'][:D], p['gate_W1'][D:], p['gate_b1'].reshape(1, -1),
            p['gate_W2'].reshape(1, -1), p['gate_b2'].reshape(1, 1),
            p['proj_W'], p['proj_b'].reshape(1, -1)]
    return pl.pallas_call(
        _final_body,
        grid=grid,
        in_specs=[pl.BlockSpec((_R_PRE, D), lambda i: (i, 0)),
                  pl.BlockSpec((_R_PRE, D), lambda i: (i, 0))] +
                 [full(a) for a in args],
        out_specs=pl.BlockSpec((_R_PRE, D), lambda i: (i, 0)),
        out_shape=jax.ShapeDtypeStruct((N, D), jnp.float32),
    )(o, states, *args)


def _att_mat(gp):
    head_of_col = jnp.arange(D)[:, None] // C == jnp.arange(H)[None, :]
    a_src = jnp.where(head_of_col, gp['att_src'].reshape(-1)[:, None], 0.0)
    a_dst = jnp.where(head_of_col, gp['att_dst'].reshape(-1)[:, None], 0.0)
    return jnp.concatenate([a_src, a_dst], axis=1).astype(jnp.float32)


def _tc_pipeline(agent_states, params, cnt):
    p = params
    role_full = jnp.tile(p['role_emb'], (1, 4))
    x = None
    for l in range(2):
        gp = p['gat'][l]
        if l == 0:
            xs = [agent_states, p['agent_emb'], role_full]
            xp, a_rows, a_t, m, x0 = _gat_pre(xs, gp['W'], _att_mat(gp))
            resid = x0
        else:
            xp, a_rows, a_t, m = _gat_pre([x], gp['W'], _att_mat(gp))
            resid = x
        x = _gat_dense(cnt, xp, a_rows, a_t, m, resid,
                       gp['bias'].reshape(1, -1), p['ln_g'][l].reshape(1, -1),
                       p['ln_b'][l].reshape(1, -1))
    qkv = _mlp(x, p)
    o = _mha(qkv)
    return _final(o, agent_states, p)


def kernel(agent_states, edge_index, params):
    cnt = _build_cnt(edge_index)
    return _tc_pipeline(agent_states, params, cnt)


# speedup vs baseline: 153.0255x; 1.0280x over previous
import functools

import jax
import jax.numpy as jnp
from jax import lax
from jax.experimental import pallas as pl
from jax.experimental.pallas import tpu as pltpu
from jax.experimental.pallas import tpu_sc as plsc

N = 4096
E = 262144
D = 128
H = 4
C = D // H


_ROWS = 16
_ROUNDS = 8
_CORE_ROWS = 2048
_STRIPS = 128
_ACC = _ROWS * N
_SLICE = E // 16
_BCAP = _SLICE + _STRIPS * 7 + 64
_EXCH = E + 16 * _STRIPS * 63 + 512
_UNROLL = 8


def _cnt_body(src_hbm, dst_hbm, zeros_hbm, neg1_hbm, out_hbm,
              d_v, s_v, pk_v, bkt_v, cnt_v, offs0_v, offs_v, raw_v, go_v,
              acc_v, exch_sh, tbl_sh):
    cid = lax.axis_index("c")
    sid = lax.axis_index("s")
    core_base = cid * _CORE_ROWS
    ebase = cid * _EXCH
    lanes = lax.broadcasted_iota(jnp.int32, (16,), 0)

    def _extract(ref, idx):
        v = ref[pl.ds((idx >> 4) * 16, 16)]
        return jnp.sum(jnp.where(lanes == (idx & 15), v, 0))

    pltpu.sync_copy(neg1_hbm, bkt_v)
    for k in range(_STRIPS // 16):
        cnt_v[pl.ds(k * 16, 16)] = jnp.zeros((16,), jnp.int32)
    base_e = sid * _SLICE
    for ch in range(4):
        off = base_e + ch * 4096
        pltpu.sync_copy(dst_hbm.at[pl.ds(off, 4096)], d_v)
        pltpu.sync_copy(src_hbm.at[pl.ds(off, 4096)], s_v)

        def p1(i, c):
            for u in range(_UNROLL):
                o = (i * _UNROLL + u) * 16
                d = d_v[pl.ds(o, 16)]
                s = s_v[pl.ds(o, 16)]
                pk_v[pl.ds(ch * 4096 + o, 16)] = d * N + s
                dr = d - core_base
                valid = (dr >= 0) & (dr < _CORE_ROWS)
                b = jnp.right_shift(dr, 4)
                cnts, last = plsc.scan_count(b, valid)
                plsc.addupdate_scatter(cnt_v, [b], cnts, mask=last)
            return c

        lax.fori_loop(0, 4096 // 16 // _UNROLL, p1, 0)
    carry = jnp.int32(0)
    for k in range(_STRIPS // 16):
        c16 = cnt_v[pl.ds(k * 16, 16)]
        lp = jnp.bitwise_and(c16 + 7, -8)
        csum = plsc.cumsum(lp)
        excl = csum - lp + carry
        offs0_v[pl.ds(k * 16, 16)] = excl
        offs_v[pl.ds(k * 16, 16)] = excl
        carry = carry + jnp.sum(lp)

    def p2(i, c):
        for u in range(_UNROLL):
            o = (i * _UNROLL + u) * 16
            p = pk_v[pl.ds(o, 16)]
            dr = jnp.right_shift(p, 12) - core_base
            valid = (dr >= 0) & (dr < _CORE_ROWS)
            b = jnp.right_shift(dr, 4)
            cnts, last = plsc.scan_count(b, valid)
            g = plsc.load_gather(offs_v, [b], mask=valid)
            plsc.store_scatter(bkt_v, [g + cnts - 1], p, mask=valid)
            plsc.addupdate_scatter(offs_v, [b], cnts, mask=last)
        return c

    lax.fori_loop(0, _SLICE // 16 // _UNROLL, p2, 0)
    pltpu.sync_copy(cnt_v, tbl_sh.at[pl.ds(sid * _STRIPS, _STRIPS)])
    plsc.subcore_barrier()
    pltpu.sync_copy(tbl_sh, raw_v)

    def lay(k, carry2):
        cc = plsc.load_gather(raw_v, [lanes * _STRIPS + k])
        pc = jnp.bitwise_and(cc + 63, -64)
        csum = plsc.cumsum(pc)
        go_v[pl.ds(k * 16, 16)] = csum - pc + carry2
        return carry2 + jnp.sum(pc)

    tot = lax.fori_loop(0, _STRIPS, lay, jnp.int32(0))
    go_v[pl.ds(16 * _STRIPS, 16)] = jnp.zeros((16,), jnp.int32) + tot

    def pub(b, c):
        cb = _extract(cnt_v, b)
        lo = pl.multiple_of(_extract(offs0_v, b), 8)
        gs = pl.multiple_of(_extract(go_v, b * 16 + sid), 64)
        trip = jnp.right_shift(cb + 63, 6)

        def pchunk(j, c2):
            pltpu.sync_copy(bkt_v.at[pl.ds(lo + j * 64, 64)],
                            exch_sh.at[pl.ds(ebase + gs + j * 64, 64)])
            return c2

        lax.fori_loop(0, trip, pchunk, 0)
        return c

    lax.fori_loop(0, _STRIPS, pub, 0)

    @pl.when(sid == 0)
    def _():
        pltpu.sync_copy(neg1_hbm.at[pl.ds(0, 512)], s_v.at[pl.ds(0, 512)])
        pltpu.sync_copy(s_v.at[pl.ds(0, 512)],
                        exch_sh.at[pl.ds(ebase + pl.multiple_of(tot, 64),
                                         512)])

    plsc.subcore_barrier()
    for r in range(_ROUNDS):
        b = sid * _ROUNDS + r
        base_flat = (core_base + sid * _STRIPS + r * _ROWS) * N
        start = pl.multiple_of(_extract(go_v, b * 16), 64)
        end = _extract(go_v, (b + 1) * 16)
        trip = jnp.right_shift(end - start + 511, 9)
        pltpu.sync_copy(zeros_hbm, acc_v)

        def cchunk(j, c2):
            pltpu.sync_copy(exch_sh.at[pl.ds(ebase + start + j * 512, 512)],
                            d_v.at[pl.ds(0, 512)])

            def cb_(i, c3):
                for u in range(_UNROLL):
                    o = (i * _UNROLL + u) * 16
                    t = d_v[pl.ds(o, 16)] - base_flat
                    m = (t >= 0) & (t < _ACC)
                    cnts, last = plsc.scan_count(t, m)
                    plsc.addupdate_scatter(acc_v, [t],
                                           cnts.astype(jnp.float32), mask=last)
                return c3

            lax.fori_loop(0, 512 // 16 // _UNROLL, cb_, 0)
            return c2

        lax.fori_loop(0, trip, cchunk, 0)
        pltpu.sync_copy(acc_v, out_hbm.at[pl.ds(base_flat, _ACC)])


def _build_cnt(edge_index):
    src = edge_index[0]
    dst = edge_index[1]
    zeros = jnp.zeros((_ACC,), jnp.float32)
    neg1 = jnp.full((_BCAP,), -1, jnp.int32)
    mesh = plsc.VectorSubcoreMesh(core_axis_name="c", subcore_axis_name="s")
    k = pl.kernel(
        _cnt_body,
        out_type=jax.ShapeDtypeStruct((N * N,), jnp.float32),
        mesh=mesh,
        compiler_params=pltpu.CompilerParams(needs_layout_passes=False),
        scratch_types=[
            pltpu.VMEM((4096,), jnp.int32),
            pltpu.VMEM((4096,), jnp.int32),
            pltpu.VMEM((_SLICE,), jnp.int32),
            pltpu.VMEM((_BCAP,), jnp.int32),
            pltpu.VMEM((_STRIPS,), jnp.int32),
            pltpu.VMEM((_STRIPS,), jnp.int32),
            pltpu.VMEM((_STRIPS,), jnp.int32),
            pltpu.VMEM((16 * _STRIPS,), jnp.int32),
            pltpu.VMEM((16 * _STRIPS + 16,), jnp.int32),
            pltpu.VMEM((_ACC,), jnp.float32),
            pltpu.HBM((2 * _EXCH,), jnp.int32),
            pltpu.VMEM_SHARED((16 * _STRIPS,), jnp.int32),
        ],
    )
    return k(src, dst, zeros, neg1).reshape(N, N)


_R_PRE = 512
_R_GAT = 512
_R_MHA = 512


def _dot(a, b):
    return jnp.dot(a, b, preferred_element_type=jnp.float32)


def _pre_body(n_add, *refs):
    i = pl.program_id(0)
    x_refs = refs[:n_add]
    w_ref, acat_ref = refs[n_add], refs[n_add + 1]
    if n_add > 1:
        xp_ref, a_ref, at_ref, m_ref, x_ref = refs[n_add + 2:]
    else:
        xp_ref, a_ref, at_ref, m_ref = refs[n_add + 2:]
    x = x_refs[0][...]
    for r in x_refs[1:]:
        x = x + r[...]
    if n_add > 1:
        x_ref[...] = x
    xp = _dot(x, w_ref[...])
    a = _dot(xp, acat_ref[...])
    xpb = xp.astype(jnp.bfloat16)
    one = jnp.ones((xp.shape[0], 1), jnp.bfloat16)
    zero = jnp.zeros((xp.shape[0], 7), jnp.bfloat16)
    parts = []
    for h in range(H):
        parts += [xpb[:, h * C:(h + 1) * C], one, zero]
    xp_ref[...] = jnp.concatenate(parts, axis=1)
    a_ref[...] = a
    at_ref[...] = a.T
    blk_max = jnp.max(a, axis=0, keepdims=True)

    @pl.when(i == 0)
    def _():
        m_ref[...] = blk_max

    @pl.when(i != 0)
    def _():
        m_ref[...] = jnp.maximum(m_ref[...], blk_max)


def _gat_pre(xs, w, a_cat):
    n_add = len(xs)
    grid = (N // _R_PRE,)
    row_spec = pl.BlockSpec((_R_PRE, D), lambda i: (i, 0))
    out_specs = [
        pl.BlockSpec((_R_PRE, D + 32), lambda i: (i, 0)),
        pl.BlockSpec((_R_PRE, 8), lambda i: (i, 0)),
        pl.BlockSpec((8, _R_PRE), lambda i: (0, i)),
        pl.BlockSpec((1, 8), lambda i: (0, 0)),
    ]
    out_shape = [
        jax.ShapeDtypeStruct((N, D + 32), jnp.bfloat16),
        jax.ShapeDtypeStruct((N, 8), jnp.float32),
        jax.ShapeDtypeStruct((8, N), jnp.float32),
        jax.ShapeDtypeStruct((1, 8), jnp.float32),
    ]
    if n_add > 1:
        out_specs.append(pl.BlockSpec((_R_PRE, D), lambda i: (i, 0)))
        out_shape.append(jax.ShapeDtypeStruct((N, D), jnp.float32))
    out = pl.pallas_call(
        functools.partial(_pre_body, n_add),
        grid=grid,
        in_specs=[row_spec] * n_add + [
            pl.BlockSpec((D, D), lambda i: (0, 0)),
            pl.BlockSpec((D, 8), lambda i: (0, 0)),
        ],
        out_specs=out_specs,
        out_shape=out_shape,
    )(*xs, w, a_cat)
    return out


def _lrelu(t):
    return jnp.where(t > 0, t, 0.2 * t)


def _gat_body(cnt_ref, xp_ref, xpt_ref, a_ref, at_ref, m_ref, res_ref,
              bias_ref, g_ref, b_ref, out_ref):
    a_blk = a_ref[...]
    ad = a_blk[:, 4:8]
    a_self = a_blk[:, 0:4]
    m_row = m_ref[...][:, 0:4]
    ast = at_ref[...][0:4, :]
    m_col = jnp.max(ast, axis=1, keepdims=True)
    shift = _lrelu(m_row + ad)
    ed_a = jnp.exp(ad + m_row - shift)
    ed_b = jnp.exp(0.2 * (ad + m_row) - shift)
    es_a = jnp.exp(ast - m_col)
    es_b = jnp.exp(0.2 * (ast - m_col))
    wdd = jnp.exp(_lrelu(a_self + ad) - shift)

    cnt = cnt_ref[...].astype(jnp.bfloat16)
    ast_b = ast.astype(jnp.bfloat16)
    nad_b = (-ad).astype(jnp.bfloat16)
    esa_b = es_a.astype(jnp.bfloat16)
    esb_b = es_b.astype(jnp.bfloat16)
    eda_b = ed_a.astype(jnp.bfloat16)
    edb_b = ed_b.astype(jnp.bfloat16)
    xpe = xp_ref[...]
    outs = []
    for h in range(H):
        msk = ast_b[h:h + 1, :] > nad_b[:, h:h + 1]
        s1 = jnp.where(msk, esa_b[h:h + 1, :], esb_b[h:h + 1, :])
        s2 = jnp.where(msk, eda_b[:, h:h + 1], edb_b[:, h:h + 1])
        w = cnt * s1 * s2
        nd = _dot(w, xpe[:, h * (C + 8):h * (C + 8) + C + 8])
        wdd_h = wdd[:, h:h + 1]
        num = nd[:, :C] + wdd_h * xpt_ref[...][:, h * (C + 8):
                                               h * (C + 8) + C].astype(
                                                   jnp.float32)
        den = nd[:, C:C + 1] + wdd_h
        outs.append(num / (den + 1e-16))
    out = jnp.concatenate(outs, axis=1) + bias_ref[...]
    mu = jnp.mean(out, axis=1, keepdims=True)
    var = jnp.mean((out - mu) ** 2, axis=1, keepdims=True)
    out = (out - mu) * lax.rsqrt(var + 1e-5) * g_ref[...] + b_ref[...]
    out_ref[...] = out + res_ref[...]


def _gat_dense(cnt, xp, a_rows, a_t, m, resid, bias, ln_g, ln_b):
    grid = (N // _R_GAT,)
    return pl.pallas_call(
        _gat_body,
        grid=grid,
        in_specs=[
            pl.BlockSpec((_R_GAT, N), lambda i: (i, 0)),
            pl.BlockSpec((N, D + 32), lambda i: (0, 0)),
            pl.BlockSpec((_R_GAT, D + 32), lambda i: (i, 0)),
            pl.BlockSpec((_R_GAT, 8), lambda i: (i, 0)),
            pl.BlockSpec((8, N), lambda i: (0, 0)),
            pl.BlockSpec((1, 8), lambda i: (0, 0)),
            pl.BlockSpec((_R_GAT, D), lambda i: (i, 0)),
            pl.BlockSpec((1, D), lambda i: (0, 0)),
            pl.BlockSpec((1, D), lambda i: (0, 0)),
            pl.BlockSpec((1, D), lambda i: (0, 0)),
        ],
        out_specs=pl.BlockSpec((_R_GAT, D), lambda i: (i, 0)),
        out_shape=jax.ShapeDtypeStruct((N, D), jnp.float32),
    )(cnt, xp, xp, a_rows, a_t, m, resid, bias, ln_g, ln_b)


def _mlp_body(x_ref, ew1, eb1, ew2, eb2, dw1, db1, dw2, db2, qw, qb, scl,
              qkv_ref):
    x = x_ref[...]
    h1 = jnp.maximum(_dot(x, ew1[...]) + eb1[...], 0.0)
    msg = _dot(h1, ew2[...]) + eb2[...]
    d1 = jnp.maximum(_dot(msg, dw1[...]) + db1[...], 0.0)
    dec = _dot(d1, dw2[...]) + db2[...]
    qkv = (_dot(dec, qw[...]) + qb[...]) * scl[...]
    qkv_ref[...] = qkv.astype(jnp.bfloat16)


def _mlp(x, p):
    grid = (N // _R_PRE,)
    full = lambda a: pl.BlockSpec(a.shape, lambda i: (0,) * a.ndim)
    scl = jnp.concatenate([jnp.full((D,), C ** -0.5, jnp.float32),
                           jnp.ones((2 * D,), jnp.float32)]).reshape(1, -1)
    args = [p['enc_W1'], p['enc_b1'].reshape(1, -1), p['enc_W2'],
            p['enc_b2'].reshape(1, -1), p['dec_W1'], p['dec_b1'].reshape(1, -1),
            p['dec_W2'], p['dec_b2'].reshape(1, -1), p['mha_in_W'],
            p['mha_in_b'].reshape(1, -1), scl]
    return pl.pallas_call(
        _mlp_body,
        grid=grid,
        in_specs=[pl.BlockSpec((_R_PRE, D), lambda i: (i, 0))] +
                 [full(a) for a in args],
        out_specs=pl.BlockSpec((_R_PRE, 3 * D), lambda i: (i, 0)),
        out_shape=jax.ShapeDtypeStruct((N, 3 * D), jnp.bfloat16),
    )(x, *args)


def _mha_body(qt_ref, kv_ref, o_ref):
    outs = []
    for h in range(H):
        q = qt_ref[...][:, h * C:(h + 1) * C]
        k = kv_ref[...][:, D + h * C:D + (h + 1) * C]
        v = kv_ref[...][:, 2 * D + h * C:2 * D + (h + 1) * C]
        scores = lax.dot_general(q, k, (((1,), (1,)), ((), ())),
                                 preferred_element_type=jnp.float32)
        p = jnp.exp(scores)
        s = jnp.sum(p, axis=1, keepdims=True)
        outs.append(_dot(p.astype(jnp.bfloat16), v) / s)
    o_ref[...] = jnp.concatenate(outs, axis=1)


def _mha(qkv):
    grid = (N // _R_MHA,)
    return pl.pallas_call(
        _mha_body,
        grid=grid,
        in_specs=[
            pl.BlockSpec((_R_MHA, 3 * D), lambda i: (i, 0)),
            pl.BlockSpec((N, 3 * D), lambda i: (0, 0)),
        ],
        out_specs=pl.BlockSpec((_R_MHA, D), lambda i: (i, 0)),
        out_shape=jax.ShapeDtypeStruct((N, D), jnp.float32),
    )(qkv, qkv)


def _final_body(o_ref, st_ref, ow, ob, gw_s, gw_a, gb1, gw2, gb2, pw, pb,
                out_ref):
    st = st_ref[...]
    agg = _dot(o_ref[...], ow[...]) + ob[...]
    g1 = jnp.maximum(_dot(st, gw_s[...]) + _dot(agg, gw_a[...]) + gb1[...], 0.0)
    logit = jnp.sum(g1 * gw2[...], axis=1, keepdims=True) + gb2[...]
    strength = 1.0 / (1.0 + jnp.exp(-logit))
    out_ref[...] = _dot(agg * strength, pw[...]) + pb[...] + st


def _final(o, states, p):
    grid = (N // _R_PRE,)
    full = lambda a: pl.BlockSpec(a.shape, lambda i: (0,) * a.ndim)
    args = [p['mha_out_W'], p['mha_out_b'].reshape(1, -1),
            p['gate_W1'][:D], p['gate_W1'][D:], p['gate_b1'].reshape(1, -1),
            p['gate_W2'].reshape(1, -1), p['gate_b2'].reshape(1, 1),
            p['proj_W'], p['proj_b'].reshape(1, -1)]
    return pl.pallas_call(
        _final_body,
        grid=grid,
        in_specs=[pl.BlockSpec((_R_PRE, D), lambda i: (i, 0)),
                  pl.BlockSpec((_R_PRE, D), lambda i: (i, 0))] +
                 [full(a) for a in args],
        out_specs=pl.BlockSpec((_R_PRE, D), lambda i: (i, 0)),
        out_shape=jax.ShapeDtypeStruct((N, D), jnp.float32),
    )(o, states, *args)


def _att_mat(gp):
    head_of_col = jnp.arange(D)[:, None] // C == jnp.arange(H)[None, :]
    a_src = jnp.where(head_of_col, gp['att_src'].reshape(-1)[:, None], 0.0)
    a_dst = jnp.where(head_of_col, gp['att_dst'].reshape(-1)[:, None], 0.0)
    return jnp.concatenate([a_src, a_dst], axis=1).astype(jnp.float32)


def _tc_pipeline(agent_states, params, cnt):
    p = params
    role_full = jnp.tile(p['role_emb'], (1, 4))
    x = None
    for l in range(2):
        gp = p['gat'][l]
        if l == 0:
            xs = [agent_states, p['agent_emb'], role_full]
            xp, a_rows, a_t, m, x0 = _gat_pre(xs, gp['W'], _att_mat(gp))
            resid = x0
        else:
            xp, a_rows, a_t, m = _gat_pre([x], gp['W'], _att_mat(gp))
            resid = x
        x = _gat_dense(cnt, xp, a_rows, a_t, m, resid,
                       gp['bias'].reshape(1, -1), p['ln_g'][l].reshape(1, -1),
                       p['ln_b'][l].reshape(1, -1))
    qkv = _mlp(x, p)
    o = _mha(qkv)
    return _final(o, agent_states, p)


def kernel(agent_states, edge_index, params):
    cnt = _build_cnt(edge_index)
    return _tc_pipeline(agent_states, params, cnt)


# speedup vs baseline: 153.7246x; 1.0046x over previous
import functools

import jax
import jax.numpy as jnp
from jax import lax
from jax.experimental import pallas as pl
from jax.experimental.pallas import tpu as pltpu
from jax.experimental.pallas import tpu_sc as plsc

N = 4096
E = 262144
D = 128
H = 4
C = D // H


_ROWS = 16
_ROUNDS = 8
_CORE_ROWS = 2048
_STRIPS = 128
_ACC = _ROWS * N
_SLICE = E // 16
_BCAP = _SLICE + _STRIPS * 7 + 64
_EXCH = E + 16 * _STRIPS * 127 + 1024
_UNROLL = 8


def _cnt_body(src_hbm, dst_hbm, zeros_hbm, neg1_hbm, out_hbm,
              d_v, s_v, pk_v, bkt_v, cnt_v, offs0_v, offs_v, raw_v, go_v,
              acc_v, exch_sh, tbl_sh):
    cid = lax.axis_index("c")
    sid = lax.axis_index("s")
    core_base = cid * _CORE_ROWS
    ebase = cid * _EXCH
    lanes = lax.broadcasted_iota(jnp.int32, (16,), 0)

    def _extract(ref, idx):
        v = ref[pl.ds((idx >> 4) * 16, 16)]
        return jnp.sum(jnp.where(lanes == (idx & 15), v, 0))

    pltpu.sync_copy(neg1_hbm, bkt_v)
    for k in range(_STRIPS // 16):
        cnt_v[pl.ds(k * 16, 16)] = jnp.zeros((16,), jnp.int32)
    base_e = sid * _SLICE
    for ch in range(4):
        off = base_e + ch * 4096
        pltpu.sync_copy(dst_hbm.at[pl.ds(off, 4096)], d_v)
        pltpu.sync_copy(src_hbm.at[pl.ds(off, 4096)], s_v)

        def p1(i, c):
            for u in range(_UNROLL):
                o = (i * _UNROLL + u) * 16
                d = d_v[pl.ds(o, 16)]
                s = s_v[pl.ds(o, 16)]
                pk_v[pl.ds(ch * 4096 + o, 16)] = d * N + s
                dr = d - core_base
                valid = (dr >= 0) & (dr < _CORE_ROWS)
                b = jnp.right_shift(dr, 4)
                cnts, last = plsc.scan_count(b, valid)
                plsc.addupdate_scatter(cnt_v, [b], cnts, mask=last)
            return c

        lax.fori_loop(0, 4096 // 16 // _UNROLL, p1, 0)
    carry = jnp.int32(0)
    for k in range(_STRIPS // 16):
        c16 = cnt_v[pl.ds(k * 16, 16)]
        lp = jnp.bitwise_and(c16 + 7, -8)
        csum = plsc.cumsum(lp)
        excl = csum - lp + carry
        offs0_v[pl.ds(k * 16, 16)] = excl
        offs_v[pl.ds(k * 16, 16)] = excl
        carry = carry + jnp.sum(lp)

    def p2(i, c):
        for u in range(_UNROLL):
            o = (i * _UNROLL + u) * 16
            p = pk_v[pl.ds(o, 16)]
            dr = jnp.right_shift(p, 12) - core_base
            valid = (dr >= 0) & (dr < _CORE_ROWS)
            b = jnp.right_shift(dr, 4)
            cnts, last = plsc.scan_count(b, valid)
            g = plsc.load_gather(offs_v, [b], mask=valid)
            plsc.store_scatter(bkt_v, [g + cnts - 1], p, mask=valid)
            plsc.addupdate_scatter(offs_v, [b], cnts, mask=last)
        return c

    lax.fori_loop(0, _SLICE // 16 // _UNROLL, p2, 0)
    pltpu.sync_copy(cnt_v, tbl_sh.at[pl.ds(sid * _STRIPS, _STRIPS)])
    plsc.subcore_barrier()
    pltpu.sync_copy(tbl_sh, raw_v)

    def lay(k, carry2):
        cc = plsc.load_gather(raw_v, [lanes * _STRIPS + k])
        pc = jnp.bitwise_and(cc + 127, -128)
        csum = plsc.cumsum(pc)
        go_v[pl.ds(k * 16, 16)] = csum - pc + carry2
        return carry2 + jnp.sum(pc)

    tot = lax.fori_loop(0, _STRIPS, lay, jnp.int32(0))
    go_v[pl.ds(16 * _STRIPS, 16)] = jnp.zeros((16,), jnp.int32) + tot

    def pub(b, c):
        cb = _extract(cnt_v, b)
        lo = pl.multiple_of(_extract(offs0_v, b), 8)
        gs = pl.multiple_of(_extract(go_v, b * 16 + sid), 128)
        trip = jnp.right_shift(cb + 127, 7)

        def pchunk(j, c2):
            pltpu.sync_copy(bkt_v.at[pl.ds(lo + j * 128, 128)],
                            exch_sh.at[pl.ds(ebase + gs + j * 128, 128)])
            return c2

        lax.fori_loop(0, trip, pchunk, 0)
        return c

    lax.fori_loop(0, _STRIPS, pub, 0)

    @pl.when(sid == 0)
    def _():
        pltpu.sync_copy(neg1_hbm.at[pl.ds(0, 1024)],
                        s_v.at[pl.ds(0, 1024)])
        pltpu.sync_copy(s_v.at[pl.ds(0, 1024)],
                        exch_sh.at[pl.ds(ebase + pl.multiple_of(tot, 128),
                                         1024)])

    plsc.subcore_barrier()
    for r in range(_ROUNDS):
        b = sid * _ROUNDS + r
        base_flat = (core_base + sid * _STRIPS + r * _ROWS) * N
        start = pl.multiple_of(_extract(go_v, b * 16), 128)
        end = _extract(go_v, (b + 1) * 16)
        trip = jnp.right_shift(end - start + 1023, 10)
        pltpu.sync_copy(zeros_hbm, acc_v)

        def cchunk(j, c2):
            pltpu.sync_copy(exch_sh.at[pl.ds(ebase + start + j * 1024, 1024)],
                            d_v.at[pl.ds(0, 1024)])

            def cb_(i, c3):
                for u in range(_UNROLL):
                    o = (i * _UNROLL + u) * 16
                    t = d_v[pl.ds(o, 16)] - base_flat
                    m = (t >= 0) & (t < _ACC)
                    cnts, last = plsc.scan_count(t, m)
                    plsc.addupdate_scatter(acc_v, [t],
                                           cnts.astype(jnp.float32), mask=last)
                return c3

            lax.fori_loop(0, 1024 // 16 // _UNROLL, cb_, 0)
            return c2

        lax.fori_loop(0, trip, cchunk, 0)
        pltpu.sync_copy(acc_v, out_hbm.at[pl.ds(base_flat, _ACC)])


def _build_cnt(edge_index):
    src = edge_index[0]
    dst = edge_index[1]
    zeros = jnp.zeros((_ACC,), jnp.float32)
    neg1 = jnp.full((_BCAP,), -1, jnp.int32)
    mesh = plsc.VectorSubcoreMesh(core_axis_name="c", subcore_axis_name="s")
    k = pl.kernel(
        _cnt_body,
        out_type=jax.ShapeDtypeStruct((N * N,), jnp.float32),
        mesh=mesh,
        compiler_params=pltpu.CompilerParams(needs_layout_passes=False),
        scratch_types=[
            pltpu.VMEM((4096,), jnp.int32),
            pltpu.VMEM((4096,), jnp.int32),
            pltpu.VMEM((_SLICE,), jnp.int32),
            pltpu.VMEM((_BCAP,), jnp.int32),
            pltpu.VMEM((_STRIPS,), jnp.int32),
            pltpu.VMEM((_STRIPS,), jnp.int32),
            pltpu.VMEM((_STRIPS,), jnp.int32),
            pltpu.VMEM((16 * _STRIPS,), jnp.int32),
            pltpu.VMEM((16 * _STRIPS + 16,), jnp.int32),
            pltpu.VMEM((_ACC,), jnp.float32),
            pltpu.HBM((2 * _EXCH,), jnp.int32),
            pltpu.VMEM_SHARED((16 * _STRIPS,), jnp.int32),
        ],
    )
    return k(src, dst, zeros, neg1).reshape(N, N)


_R_PRE = 512
_R_GAT = 512
_R_MHA = 512


def _dot(a, b):
    return jnp.dot(a, b, preferred_element_type=jnp.float32)


def _pre_body(n_add, *refs):
    i = pl.program_id(0)
    x_refs = refs[:n_add]
    w_ref, acat_ref = refs[n_add], refs[n_add + 1]
    if n_add > 1:
        xp_ref, a_ref, at_ref, m_ref, x_ref = refs[n_add + 2:]
    else:
        xp_ref, a_ref, at_ref, m_ref = refs[n_add + 2:]
    x = x_refs[0][...]
    for r in x_refs[1:]:
        x = x + r[...]
    if n_add > 1:
        x_ref[...] = x
    xp = _dot(x, w_ref[...])
    a = _dot(xp, acat_ref[...])
    xpb = xp.astype(jnp.bfloat16)
    one = jnp.ones((xp.shape[0], 1), jnp.bfloat16)
    zero = jnp.zeros((xp.shape[0], 7), jnp.bfloat16)
    parts = []
    for h in range(H):
        parts += [xpb[:, h * C:(h + 1) * C], one, zero]
    xp_ref[...] = jnp.concatenate(parts, axis=1)
    a_ref[...] = a
    at_ref[...] = a.T
    blk_max = jnp.max(a, axis=0, keepdims=True)

    @pl.when(i == 0)
    def _():
        m_ref[...] = blk_max

    @pl.when(i != 0)
    def _():
        m_ref[...] = jnp.maximum(m_ref[...], blk_max)


def _gat_pre(xs, w, a_cat):
    n_add = len(xs)
    grid = (N // _R_PRE,)
    row_spec = pl.BlockSpec((_R_PRE, D), lambda i: (i, 0))
    out_specs = [
        pl.BlockSpec((_R_PRE, D + 32), lambda i: (i, 0)),
        pl.BlockSpec((_R_PRE, 8), lambda i: (i, 0)),
        pl.BlockSpec((8, _R_PRE), lambda i: (0, i)),
        pl.BlockSpec((1, 8), lambda i: (0, 0)),
    ]
    out_shape = [
        jax.ShapeDtypeStruct((N, D + 32), jnp.bfloat16),
        jax.ShapeDtypeStruct((N, 8), jnp.float32),
        jax.ShapeDtypeStruct((8, N), jnp.float32),
        jax.ShapeDtypeStruct((1, 8), jnp.float32),
    ]
    if n_add > 1:
        out_specs.append(pl.BlockSpec((_R_PRE, D), lambda i: (i, 0)))
        out_shape.append(jax.ShapeDtypeStruct((N, D), jnp.float32))
    out = pl.pallas_call(
        functools.partial(_pre_body, n_add),
        grid=grid,
        in_specs=[row_spec] * n_add + [
            pl.BlockSpec((D, D), lambda i: (0, 0)),
            pl.BlockSpec((D, 8), lambda i: (0, 0)),
        ],
        out_specs=out_specs,
        out_shape=out_shape,
    )(*xs, w, a_cat)
    return out


def _lrelu(t):
    return jnp.where(t > 0, t, 0.2 * t)


def _gat_body(cnt_ref, xp_ref, xpt_ref, a_ref, at_ref, m_ref, res_ref,
              bias_ref, g_ref, b_ref, out_ref):
    a_blk = a_ref[...]
    ad = a_blk[:, 4:8]
    a_self = a_blk[:, 0:4]
    m_row = m_ref[...][:, 0:4]
    ast = at_ref[...][0:4, :]
    m_col = jnp.max(ast, axis=1, keepdims=True)
    shift = _lrelu(m_row + ad)
    ed_a = jnp.exp(ad + m_row - shift)
    ed_b = jnp.exp(0.2 * (ad + m_row) - shift)
    es_a = jnp.exp(ast - m_col)
    es_b = jnp.exp(0.2 * (ast - m_col))
    wdd = jnp.exp(_lrelu(a_self + ad) - shift)

    cnt = cnt_ref[...].astype(jnp.bfloat16)
    ast_b = ast.astype(jnp.bfloat16)
    nad_b = (-ad).astype(jnp.bfloat16)
    esa_b = es_a.astype(jnp.bfloat16)
    esb_b = es_b.astype(jnp.bfloat16)
    eda_b = ed_a.astype(jnp.bfloat16)
    edb_b = ed_b.astype(jnp.bfloat16)
    xpe = xp_ref[...]
    outs = []
    for h in range(H):
        msk = ast_b[h:h + 1, :] > nad_b[:, h:h + 1]
        s1 = jnp.where(msk, esa_b[h:h + 1, :], esb_b[h:h + 1, :])
        s2 = jnp.where(msk, eda_b[:, h:h + 1], edb_b[:, h:h + 1])
        w = cnt * s1 * s2
        nd = _dot(w, xpe[:, h * (C + 8):h * (C + 8) + C + 8])
        wdd_h = wdd[:, h:h + 1]
        num = nd[:, :C] + wdd_h * xpt_ref[...][:, h * (C + 8):
                                               h * (C + 8) + C].astype(
                                                   jnp.float32)
        den = nd[:, C:C + 1] + wdd_h
        outs.append(num / (den + 1e-16))
    out = jnp.concatenate(outs, axis=1) + bias_ref[...]
    mu = jnp.mean(out, axis=1, keepdims=True)
    var = jnp.mean((out - mu) ** 2, axis=1, keepdims=True)
    out = (out - mu) * lax.rsqrt(var + 1e-5) * g_ref[...] + b_ref[...]
    out_ref[...] = out + res_ref[...]


def _gat_dense(cnt, xp, a_rows, a_t, m, resid, bias, ln_g, ln_b):
    grid = (N // _R_GAT,)
    return pl.pallas_call(
        _gat_body,
        grid=grid,
        in_specs=[
            pl.BlockSpec((_R_GAT, N), lambda i: (i, 0)),
            pl.BlockSpec((N, D + 32), lambda i: (0, 0)),
            pl.BlockSpec((_R_GAT, D + 32), lambda i: (i, 0)),
            pl.BlockSpec((_R_GAT, 8), lambda i: (i, 0)),
            pl.BlockSpec((8, N), lambda i: (0, 0)),
            pl.BlockSpec((1, 8), lambda i: (0, 0)),
            pl.BlockSpec((_R_GAT, D), lambda i: (i, 0)),
            pl.BlockSpec((1, D), lambda i: (0, 0)),
            pl.BlockSpec((1, D), lambda i: (0, 0)),
            pl.BlockSpec((1, D), lambda i: (0, 0)),
        ],
        out_specs=pl.BlockSpec((_R_GAT, D), lambda i: (i, 0)),
        out_shape=jax.ShapeDtypeStruct((N, D), jnp.float32),
    )(cnt, xp, xp, a_rows, a_t, m, resid, bias, ln_g, ln_b)


def _mlp_body(x_ref, ew1, eb1, ew2, eb2, dw1, db1, dw2, db2, qw, qb, scl,
              qkv_ref):
    x = x_ref[...]
    h1 = jnp.maximum(_dot(x, ew1[...]) + eb1[...], 0.0)
    msg = _dot(h1, ew2[...]) + eb2[...]
    d1 = jnp.maximum(_dot(msg, dw1[...]) + db1[...], 0.0)
    dec = _dot(d1, dw2[...]) + db2[...]
    qkv = (_dot(dec, qw[...]) + qb[...]) * scl[...]
    qkv_ref[...] = qkv.astype(jnp.bfloat16)


def _mlp(x, p):
    grid = (N // _R_PRE,)
    full = lambda a: pl.BlockSpec(a.shape, lambda i: (0,) * a.ndim)
    scl = jnp.concatenate([jnp.full((D,), C ** -0.5, jnp.float32),
                           jnp.ones((2 * D,), jnp.float32)]).reshape(1, -1)
    args = [p['enc_W1'], p['enc_b1'].reshape(1, -1), p['enc_W2'],
            p['enc_b2'].reshape(1, -1), p['dec_W1'], p['dec_b1'].reshape(1, -1),
            p['dec_W2'], p['dec_b2'].reshape(1, -1), p['mha_in_W'],
            p['mha_in_b'].reshape(1, -1), scl]
    return pl.pallas_call(
        _mlp_body,
        grid=grid,
        in_specs=[pl.BlockSpec((_R_PRE, D), lambda i: (i, 0))] +
                 [full(a) for a in args],
        out_specs=pl.BlockSpec((_R_PRE, 3 * D), lambda i: (i, 0)),
        out_shape=jax.ShapeDtypeStruct((N, 3 * D), jnp.bfloat16),
    )(x, *args)


def _mha_body(qt_ref, kv_ref, o_ref):
    outs = []
    for h in range(H):
        q = qt_ref[...][:, h * C:(h + 1) * C]
        k = kv_ref[...][:, D + h * C:D + (h + 1) * C]
        v = kv_ref[...][:, 2 * D + h * C:2 * D + (h + 1) * C]
        scores = lax.dot_general(q, k, (((1,), (1,)), ((), ())),
                                 preferred_element_type=jnp.float32)
        p = jnp.exp(scores)
        s = jnp.sum(p, axis=1, keepdims=True)
        outs.append(_dot(p.astype(jnp.bfloat16), v) / s)
    o_ref[...] = jnp.concatenate(outs, axis=1)


def _mha(qkv):
    grid = (N // _R_MHA,)
    return pl.pallas_call(
        _mha_body,
        grid=grid,
        in_specs=[
            pl.BlockSpec((_R_MHA, 3 * D), lambda i: (i, 0)),
            pl.BlockSpec((N, 3 * D), lambda i: (0, 0)),
        ],
        out_specs=pl.BlockSpec((_R_MHA, D), lambda i: (i, 0)),
        out_shape=jax.ShapeDtypeStruct((N, D), jnp.float32),
    )(qkv, qkv)


def _final_body(o_ref, st_ref, ow, ob, gw_s, gw_a, gb1, gw2, gb2, pw, pb,
                out_ref):
    st = st_ref[...]
    agg = _dot(o_ref[...], ow[...]) + ob[...]
    g1 = jnp.maximum(_dot(st, gw_s[...]) + _dot(agg, gw_a[...]) + gb1[...], 0.0)
    logit = jnp.sum(g1 * gw2[...], axis=1, keepdims=True) + gb2[...]
    strength = 1.0 / (1.0 + jnp.exp(-logit))
    out_ref[...] = _dot(agg * strength, pw[...]) + pb[...] + st


def _final(o, states, p):
    grid = (N // _R_PRE,)
    full = lambda a: pl.BlockSpec(a.shape, lambda i: (0,) * a.ndim)
    args = [p['mha_out_W'], p['mha_out_b'].reshape(1, -1),
            p['gate_W1'][:D], p['gate_W1'][D:], p['gate_b1'].reshape(1, -1),
            p['gate_W2'].reshape(1, -1), p['gate_b2'].reshape(1, 1),
            p['proj_W'], p['proj_b'].reshape(1, -1)]
    return pl.pallas_call(
        _final_body,
        grid=grid,
        in_specs=[pl.BlockSpec((_R_PRE, D), lambda i: (i, 0)),
                  pl.BlockSpec((_R_PRE, D), lambda i: (i, 0))] +
                 [full(a) for a in args],
        out_specs=pl.BlockSpec((_R_PRE, D), lambda i: (i, 0)),
        out_shape=jax.ShapeDtypeStruct((N, D), jnp.float32),
    )(o, states, *args)


def _att_mat(gp):
    head_of_col = jnp.arange(D)[:, None] // C == jnp.arange(H)[None, :]
    a_src = jnp.where(head_of_col, gp['att_src'].reshape(-1)[:, None], 0.0)
    a_dst = jnp.where(head_of_col, gp['att_dst'].reshape(-1)[:, None], 0.0)
    return jnp.concatenate([a_src, a_dst], axis=1).astype(jnp.float32)


def _tc_pipeline(agent_states, params, cnt):
    p = params
    role_full = jnp.tile(p['role_emb'], (1, 4))
    x = None
    for l in range(2):
        gp = p['gat'][l]
        if l == 0:
            xs = [agent_states, p['agent_emb'], role_full]
            xp, a_rows, a_t, m, x0 = _gat_pre(xs, gp['W'], _att_mat(gp))
            resid = x0
        else:
            xp, a_rows, a_t, m = _gat_pre([x], gp['W'], _att_mat(gp))
            resid = x
        x = _gat_dense(cnt, xp, a_rows, a_t, m, resid,
                       gp['bias'].reshape(1, -1), p['ln_g'][l].reshape(1, -1),
                       p['ln_b'][l].reshape(1, -1))
    qkv = _mlp(x, p)
    o = _mha(qkv)
    return _final(o, agent_states, p)


def kernel(agent_states, edge_index, params):
    cnt = _build_cnt(edge_index)
    return _tc_pipeline(agent_states, params, cnt)
